# Initial kernel scaffold; baseline (speedup 1.0000x reference)
import jax
import jax.numpy as jnp
from jax.experimental import pallas as pl


def kernel(x, edge_index, edge_type, comp0, bases0, root0, bias0, bn_gamma, bn_beta, bn_mean, bn_var, comp1, bases1, root1, bias1):
    raise NotImplementedError("write your pallas kernel here")


# speedup vs baseline: 10.5089x; 10.5089x over previous
import functools

import jax
import jax.numpy as jnp
from jax import lax
from jax.experimental import pallas as pl
from jax.experimental.pallas import tpu as pltpu
from jax.experimental.pallas import tpu_sc as plsc

N = 10000
E = 320000
C = 128
R = 5
NB = 4
EPS_BN = 1e-5
EPS_NORM = 1e-12

NC = 2
NS = 16
L = 16
NW = NC * NS
CK = 128
NCHG = E // CK
NCH_BASE = NCHG // NW
NCH_REM = NCHG % NW
CPAD = 51200
CPT = CPAD // NS
ZPT = 624
NBK = 1000
GRID = N // NBK

_mesh = plsc.VectorSubcoreMesh(core_axis_name="c", subcore_axis_name="s")
_sc_params = pltpu.CompilerParams(needs_layout_passes=False)


@functools.partial(
    pl.kernel,
    out_type=(
        jax.ShapeDtypeStruct((2 * CPAD,), jnp.float32),
        jax.ShapeDtypeStruct((E,), jnp.int32),
        jax.ShapeDtypeStruct((E,), jnp.int32),
    ),
    mesh=_mesh,
    compiler_params=_sc_params,
    scratch_types=[
        pltpu.VMEM((CK,), jnp.int32),
        pltpu.VMEM((CK,), jnp.int32),
        pltpu.VMEM((CK,), jnp.int32),
        pltpu.VMEM((CK,), jnp.int32),
        pltpu.VMEM((CK,), jnp.int32),
        pltpu.VMEM((CK,), jnp.float32),
        pltpu.VMEM((CPT,), jnp.float32),
        pltpu.VMEM_SHARED((CPAD,), jnp.float32),
    ],
)
def _counts_sc(src_hbm, rt_hbm, dst_hbm, cnt_hbm, gidx_hbm, cidx_hbm,
               src_v, rt_v, dst_v, g_v, ci_v, ones_v, zer_v, acc_sh):
    cid = lax.axis_index("c")
    sid = lax.axis_index("s")
    wid = sid * NC + cid

    def fill_ones(i, _):
        ones_v[pl.ds(i * L, L)] = jnp.full((L,), 1.0, jnp.float32)
        return 0
    lax.fori_loop(0, CK // L, fill_ones, 0)

    def fill_zeros(i, _):
        zer_v[pl.ds(i * L, L)] = jnp.zeros((L,), jnp.float32)
        return 0
    lax.fori_loop(0, CPT // L, fill_zeros, 0)

    pltpu.sync_copy(zer_v, acc_sh.at[pl.ds(sid * CPT, CPT)])
    plsc.subcore_barrier()

    def chunk(i, _):
        off = (wid + i * NW) * CK
        pltpu.sync_copy(src_hbm.at[pl.ds(off, CK)], src_v)
        pltpu.sync_copy(rt_hbm.at[pl.ds(off, CK)], rt_v)
        pltpu.sync_copy(dst_hbm.at[pl.ds(off, CK)], dst_v)
        for j in range(CK // L):
            sl = pl.ds(j * L, L)
            s16 = src_v[sl]
            r16 = rt_v[sl]
            d16 = dst_v[sl]
            g_v[sl] = r16 * N + s16
            ci_v[sl] = d16 * R + r16
        pltpu.sync_copy(g_v, gidx_hbm.at[pl.ds(off, CK)])
        pltpu.sync_copy(ci_v, cidx_hbm.at[pl.ds(off, CK)])
        pltpu.sync_copy(ones_v, acc_sh.at[ci_v], add=True)
        return 0
    nloc = NCH_BASE + jnp.where(wid < NCH_REM, 1, 0)
    lax.fori_loop(0, nloc, chunk, 0)

    plsc.subcore_barrier()
    pltpu.sync_copy(acc_sh.at[pl.ds(sid * CPT, CPT)],
                    cnt_hbm.at[pl.ds(cid * CPAD + sid * CPT, CPT)])


@functools.partial(
    pl.kernel,
    out_type=jax.ShapeDtypeStruct((E,), jnp.float32),
    mesh=_mesh,
    compiler_params=_sc_params,
    scratch_types=[
        pltpu.VMEM((CPAD,), jnp.float32),
        pltpu.VMEM((CPAD,), jnp.float32),
        pltpu.VMEM((CK,), jnp.int32),
        pltpu.VMEM((CK,), jnp.float32),
    ],
)
def _weights_sc(cnt_hbm, cidx_hbm, w_hbm, p0_v, p1_v, ci_v, w_v):
    wid = lax.axis_index("s") * NC + lax.axis_index("c")
    pltpu.sync_copy(cnt_hbm.at[pl.ds(0, CPAD)], p0_v)
    pltpu.sync_copy(cnt_hbm.at[pl.ds(CPAD, CPAD)], p1_v)

    def chunk(i, _):
        off = (wid + i * NW) * CK
        pltpu.sync_copy(cidx_hbm.at[pl.ds(off, CK)], ci_v)
        for j in range(CK // L):
            sl = pl.ds(j * L, L)
            idx16 = ci_v[sl]
            c = plsc.load_gather(p0_v, [idx16]) + plsc.load_gather(p1_v, [idx16])
            w_v[sl] = 1.0 / jnp.maximum(c, 1.0)
        pltpu.sync_copy(w_v, w_hbm.at[pl.ds(off, CK)])
        return 0
    nloc = NCH_BASE + jnp.where(wid < NCH_REM, 1, 0)
    lax.fori_loop(0, nloc, chunk, 0)


@functools.partial(
    pl.kernel,
    out_type=jax.ShapeDtypeStruct((2 * N, C), jnp.float32),
    mesh=_mesh,
    compiler_params=_sc_params,
    scratch_types=[
        pltpu.VMEM((CK,), jnp.int32),
        pltpu.VMEM((CK,), jnp.int32),
        pltpu.VMEM((CK,), jnp.float32),
        pltpu.VMEM((CK, C), jnp.float32),
        pltpu.SemaphoreType.DMA,
        pltpu.VMEM_SHARED((N, C), jnp.float32),
    ],
)
def _agg_sc(z_hbm, gidx_hbm, dst_hbm, w_hbm, out_hbm,
            g_v, d_v, w_v, rows_v, sem, acc_sh):
    cid = lax.axis_index("c")
    sid = lax.axis_index("s")
    wid = sid * NC + cid

    def zero_rows(i, _):
        for j in range(C // L):
            rows_v[i, pl.ds(j * L, L)] = jnp.zeros((L,), jnp.float32)
        return 0
    lax.fori_loop(0, CK, zero_rows, 0)

    zb = sid * ZPT
    def zero_acc(i, _):
        pltpu.sync_copy(rows_v, acc_sh.at[pl.ds(zb + i * CK, CK)])
        return 0
    lax.fori_loop(0, ZPT // CK, zero_acc, 0)
    pltpu.sync_copy(rows_v.at[pl.ds(0, ZPT % CK)],
                    acc_sh.at[pl.ds(zb + (ZPT // CK) * CK, ZPT % CK)])
    @pl.when(sid == 0)
    def _():
        pltpu.sync_copy(rows_v.at[pl.ds(0, N - NS * ZPT)],
                        acc_sh.at[pl.ds(NS * ZPT, N - NS * ZPT)])
    plsc.subcore_barrier()

    def chunk(i, _):
        off = (wid + i * NW) * CK
        pltpu.sync_copy(gidx_hbm.at[pl.ds(off, CK)], g_v)
        pltpu.sync_copy(dst_hbm.at[pl.ds(off, CK)], d_v)
        pltpu.sync_copy(w_hbm.at[pl.ds(off, CK)], w_v)
        pltpu.async_copy(z_hbm.at[g_v], rows_v, sem).wait()

        def scale(e, _):
            w16 = plsc.load_gather(w_v, [jnp.full((L,), 0, jnp.int32) + e])
            for j in range(C // L):
                sl = pl.ds(j * L, L)
                rows_v[e, sl] = rows_v[e, sl] * w16
            return 0
        lax.fori_loop(0, CK, scale, 0)

        pltpu.sync_copy(rows_v, acc_sh.at[d_v], add=True)
        return 0
    nloc = NCH_BASE + jnp.where(wid < NCH_REM, 1, 0)
    lax.fori_loop(0, nloc, chunk, 0)

    plsc.subcore_barrier()
    ob = cid * N
    def writeout(i, _):
        pltpu.sync_copy(acc_sh.at[pl.ds(sid * ZPT + i * CK, CK)],
                        out_hbm.at[pl.ds(ob + sid * ZPT + i * CK, CK)])
        return 0
    lax.fori_loop(0, ZPT // CK, writeout, 0)
    pltpu.sync_copy(acc_sh.at[pl.ds(sid * ZPT + (ZPT // CK) * CK, ZPT % CK)],
                    out_hbm.at[pl.ds(ob + sid * ZPT + (ZPT // CK) * CK, ZPT % CK)])
    @pl.when(sid == 0)
    def _():
        pltpu.sync_copy(acc_sh.at[pl.ds(NS * ZPT, N - NS * ZPT)],
                        out_hbm.at[pl.ds(ob + NS * ZPT, N - NS * ZPT)])


def _prep_tc_body(comp0_ref, b0_ref, comp1_ref, b1_ref, g_ref, be_ref, m_ref,
                  v_ref, w0_ref, w1_ref, ab_ref):
    w0_ref[...] = jnp.dot(comp0_ref[...], b0_ref[...],
                          preferred_element_type=jnp.float32)
    w1_ref[...] = jnp.dot(comp1_ref[...], b1_ref[...],
                          preferred_element_type=jnp.float32)
    a = g_ref[...] * lax.rsqrt(v_ref[...] + EPS_BN)
    ab_ref[0:1, :] = a
    ab_ref[1:2, :] = be_ref[...] - m_ref[...] * a


_prep_tc = pl.pallas_call(
    _prep_tc_body,
    out_shape=(
        jax.ShapeDtypeStruct((R, C * C), jnp.float32),
        jax.ShapeDtypeStruct((R, C * C), jnp.float32),
        jax.ShapeDtypeStruct((2, C), jnp.float32),
    ),
)


def _proj_tc_body(h_ref, w_ref, root_ref, bias_ref, z_ref, base_ref):
    h = h_ref[...]
    for r in range(R):
        z_ref[r] = jnp.dot(h, w_ref[r], preferred_element_type=jnp.float32)
    base_ref[...] = jnp.dot(h, root_ref[...],
                            preferred_element_type=jnp.float32) + bias_ref[...]


_proj_tc = pl.pallas_call(
    _proj_tc_body,
    grid=(GRID,),
    in_specs=[
        pl.BlockSpec((NBK, C), lambda i: (i, 0)),
        pl.BlockSpec((R, C, C), lambda i: (0, 0, 0)),
        pl.BlockSpec((C, C), lambda i: (0, 0)),
        pl.BlockSpec((1, C), lambda i: (0, 0)),
    ],
    out_specs=(
        pl.BlockSpec((R, NBK, C), lambda i: (0, i, 0)),
        pl.BlockSpec((NBK, C), lambda i: (i, 0)),
    ),
    out_shape=(
        jax.ShapeDtypeStruct((R, N, C), jnp.float32),
        jax.ShapeDtypeStruct((N, C), jnp.float32),
    ),
)


def _proj2_tc_body(base0_ref, p0_ref, p1_ref, ab_ref, w_ref, root_ref,
                   bias_ref, z_ref, base_ref):
    y = base0_ref[...] + p0_ref[...] + p1_ref[...]
    h = jnp.maximum(y * ab_ref[0:1, :] + ab_ref[1:2, :], 0.0)
    for r in range(R):
        z_ref[r] = jnp.dot(h, w_ref[r], preferred_element_type=jnp.float32)
    base_ref[...] = jnp.dot(h, root_ref[...],
                            preferred_element_type=jnp.float32) + bias_ref[...]


_proj2_tc = pl.pallas_call(
    _proj2_tc_body,
    grid=(GRID,),
    in_specs=[
        pl.BlockSpec((NBK, C), lambda i: (i, 0)),
        pl.BlockSpec((NBK, C), lambda i: (i, 0)),
        pl.BlockSpec((NBK, C), lambda i: (i, 0)),
        pl.BlockSpec((2, C), lambda i: (0, 0)),
        pl.BlockSpec((R, C, C), lambda i: (0, 0, 0)),
        pl.BlockSpec((C, C), lambda i: (0, 0)),
        pl.BlockSpec((1, C), lambda i: (0, 0)),
    ],
    out_specs=(
        pl.BlockSpec((R, NBK, C), lambda i: (0, i, 0)),
        pl.BlockSpec((NBK, C), lambda i: (i, 0)),
    ),
    out_shape=(
        jax.ShapeDtypeStruct((R, N, C), jnp.float32),
        jax.ShapeDtypeStruct((N, C), jnp.float32),
    ),
)


def _final_tc_body(base_ref, p0_ref, p1_ref, out_ref):
    y = base_ref[...] + p0_ref[...] + p1_ref[...]
    nrm = jnp.sqrt(jnp.sum(y * y, axis=-1, keepdims=True))
    out_ref[...] = y / jnp.maximum(nrm, EPS_NORM)


_final_tc = pl.pallas_call(
    _final_tc_body,
    grid=(GRID,),
    in_specs=[
        pl.BlockSpec((NBK, C), lambda i: (i, 0)),
        pl.BlockSpec((NBK, C), lambda i: (i, 0)),
        pl.BlockSpec((NBK, C), lambda i: (i, 0)),
    ],
    out_specs=pl.BlockSpec((NBK, C), lambda i: (i, 0)),
    out_shape=jax.ShapeDtypeStruct((N, C), jnp.float32),
)


def kernel(x, edge_index, edge_type, comp0, bases0, root0, bias0,
           bn_gamma, bn_beta, bn_mean, bn_var, comp1, bases1, root1, bias1):
    src = edge_index[0].astype(jnp.int32)
    dst = edge_index[1].astype(jnp.int32)
    rt = edge_type.astype(jnp.int32)

    w0f, w1f, ab = _prep_tc(comp0, bases0.reshape(NB, C * C),
                            comp1, bases1.reshape(NB, C * C),
                            bn_gamma.reshape(1, C), bn_beta.reshape(1, C),
                            bn_mean.reshape(1, C), bn_var.reshape(1, C))
    w0 = w0f.reshape(R, C, C)
    w1 = w1f.reshape(R, C, C)

    cnt, gidx, cidx = _counts_sc(src, rt, dst)
    w = _weights_sc(cnt, cidx)

    z0, base0 = _proj_tc(x, w0, root0, bias0.reshape(1, C))
    parts0 = _agg_sc(z0.reshape(R * N, C), gidx, dst, w)

    z1, base1 = _proj2_tc(base0, parts0[:N], parts0[N:], ab, w1, root1,
                          bias1.reshape(1, C))
    parts1 = _agg_sc(z1.reshape(R * N, C), gidx, dst, w)

    return _final_tc(base1, parts1[:N], parts1[N:])


# speedup vs baseline: 17.9927x; 1.7121x over previous
import functools

import jax
import jax.numpy as jnp
from jax import lax
from jax.experimental import pallas as pl
from jax.experimental.pallas import tpu as pltpu
from jax.experimental.pallas import tpu_sc as plsc

N = 10000
E = 320000
C = 128
R = 5
NB = 4
EPS_BN = 1e-5
EPS_NORM = 1e-12

NC = 2
NS = 16
L = 16
NW = NC * NS
CK = 128
NCHG = E // CK
NCT = NCHG // NW
TAIL = NCHG - NW * NCT
CPAD = 51200
CPT = CPAD // NS
ZPT = 624
NBK = 1000
GRID = N // NBK

_mesh = plsc.VectorSubcoreMesh(core_axis_name="c", subcore_axis_name="s")
_sc_params = pltpu.CompilerParams(needs_layout_passes=False)


@functools.partial(
    pl.kernel,
    out_type=(
        jax.ShapeDtypeStruct((2 * CPAD,), jnp.float32),
        jax.ShapeDtypeStruct((E,), jnp.int32),
        jax.ShapeDtypeStruct((E,), jnp.int32),
    ),
    mesh=_mesh,
    compiler_params=_sc_params,
    scratch_types=[
        pltpu.VMEM((2, CK), jnp.int32),
        pltpu.VMEM((2, CK), jnp.int32),
        pltpu.VMEM((2, CK), jnp.int32),
        pltpu.VMEM((CK,), jnp.int32),
        pltpu.VMEM((CK,), jnp.int32),
        pltpu.VMEM((CK,), jnp.int32),
        pltpu.VMEM((CK,), jnp.int32),
        pltpu.VMEM((CK,), jnp.float32),
        pltpu.VMEM((CPT,), jnp.float32),
        pltpu.SemaphoreType.DMA,
        pltpu.SemaphoreType.DMA,
        pltpu.SemaphoreType.DMA,
        pltpu.SemaphoreType.DMA,
        pltpu.VMEM_SHARED((CPAD,), jnp.float32),
    ],
)
def _counts_sc(src_hbm, rt_hbm, dst_hbm, cnt_hbm, gidx_hbm, cidx_hbm,
               src_v, rt_v, dst_v, g0, g1, c0, c1, ones_v, zer_v,
               isem0, isem1, wsem0, wsem1, acc_sh):
    cid = lax.axis_index("c")
    sid = lax.axis_index("s")
    wid = sid * NC + cid
    gbuf = (g0, g1)
    cbuf = (c0, c1)
    isem = (isem0, isem1)
    wsem = (wsem0, wsem1)

    def fill_ones(i, _):
        ones_v[pl.ds(i * L, L)] = jnp.full((L,), 1.0, jnp.float32)
        return 0
    lax.fori_loop(0, CK // L, fill_ones, 0)

    def fill_zeros(i, _):
        zer_v[pl.ds(i * L, L)] = jnp.zeros((L,), jnp.float32)
        return 0
    lax.fori_loop(0, CPT // L, fill_zeros, 0)

    pltpu.sync_copy(zer_v, acc_sh.at[pl.ds(sid * CPT, CPT)])
    plsc.subcore_barrier()

    def cg(i):
        return (wid + jnp.minimum(i, NCT - 1) * NW) * CK

    def fire_idx(i, b):
        off = cg(i)
        pltpu.async_copy(src_hbm.at[pl.ds(off, CK)], src_v.at[b], isem[b])
        pltpu.async_copy(rt_hbm.at[pl.ds(off, CK)], rt_v.at[b], isem[b])
        pltpu.async_copy(dst_hbm.at[pl.ds(off, CK)], dst_v.at[b], isem[b])

    def wait_idx(i, b):
        off = cg(i)
        pltpu.make_async_copy(src_hbm.at[pl.ds(off, CK)], src_v.at[b], isem[b]).wait()
        pltpu.make_async_copy(rt_hbm.at[pl.ds(off, CK)], rt_v.at[b], isem[b]).wait()
        pltpu.make_async_copy(dst_hbm.at[pl.ds(off, CK)], dst_v.at[b], isem[b]).wait()

    def wait_writes(i, b):
        off = cg(i)
        pltpu.make_async_copy(gbuf[b], gidx_hbm.at[pl.ds(off, CK)], wsem[b]).wait()
        pltpu.make_async_copy(cbuf[b], cidx_hbm.at[pl.ds(off, CK)], wsem[b]).wait()

    def compute(b):
        for j in range(CK // L):
            sl = pl.ds(j * L, L)
            s16 = src_v[b, sl]
            r16 = rt_v[b, sl]
            d16 = dst_v[b, sl]
            gbuf[b][sl] = r16 * N + s16
            cbuf[b][sl] = d16 * R + r16

    def step(i, b, first):
        wait_idx(i, b)
        fire_idx(i + 1, 1 - b)
        if first:
            pass
        else:
            @pl.when(i >= 2)
            def _():
                wait_writes(i - 2, b)
        compute(b)
        off = cg(i)
        pltpu.async_copy(gbuf[b], gidx_hbm.at[pl.ds(off, CK)], wsem[b])
        pltpu.async_copy(cbuf[b], cidx_hbm.at[pl.ds(off, CK)], wsem[b])
        pltpu.sync_copy(ones_v, acc_sh.at[cbuf[b]], add=True)

    fire_idx(0, 0)

    def pair(k, _):
        i = k * 2
        step(i, 0, False)
        step(i + 1, 1, False)
        return 0
    lax.fori_loop(0, NCT // 2, pair, 0)

    wait_idx(NCT, 0)
    wait_writes(NCT - 2, 0)
    wait_writes(NCT - 1, 1)

    @pl.when(wid < TAIL)
    def _():
        off = (NW * NCT + wid) * CK
        pltpu.sync_copy(src_hbm.at[pl.ds(off, CK)], src_v.at[0])
        pltpu.sync_copy(rt_hbm.at[pl.ds(off, CK)], rt_v.at[0])
        pltpu.sync_copy(dst_hbm.at[pl.ds(off, CK)], dst_v.at[0])
        compute(0)
        pltpu.sync_copy(gbuf[0], gidx_hbm.at[pl.ds(off, CK)])
        pltpu.sync_copy(cbuf[0], cidx_hbm.at[pl.ds(off, CK)])
        pltpu.sync_copy(ones_v, acc_sh.at[cbuf[0]], add=True)

    plsc.subcore_barrier()
    pltpu.sync_copy(acc_sh.at[pl.ds(sid * CPT, CPT)],
                    cnt_hbm.at[pl.ds(cid * CPAD + sid * CPT, CPT)])


@functools.partial(
    pl.kernel,
    out_type=jax.ShapeDtypeStruct((E,), jnp.float32),
    mesh=_mesh,
    compiler_params=_sc_params,
    scratch_types=[
        pltpu.VMEM((CPAD,), jnp.float32),
        pltpu.VMEM((CPAD,), jnp.float32),
        pltpu.VMEM((2, CK), jnp.int32),
        pltpu.VMEM((CK,), jnp.float32),
        pltpu.VMEM((CK,), jnp.float32),
        pltpu.SemaphoreType.DMA,
        pltpu.SemaphoreType.DMA,
        pltpu.SemaphoreType.DMA,
        pltpu.SemaphoreType.DMA,
    ],
)
def _weights_sc(cnt_hbm, cidx_hbm, w_hbm, p0_v, p1_v, ci_v, w0, w1,
                isem0, isem1, wsem0, wsem1):
    wid = lax.axis_index("s") * NC + lax.axis_index("c")
    wbuf = (w0, w1)
    isem = (isem0, isem1)
    wsem = (wsem0, wsem1)
    pltpu.sync_copy(cnt_hbm.at[pl.ds(0, CPAD)], p0_v)
    pltpu.sync_copy(cnt_hbm.at[pl.ds(CPAD, CPAD)], p1_v)

    def cg(i):
        return (wid + jnp.minimum(i, NCT - 1) * NW) * CK

    def step(i, b):
        off = cg(i)
        pltpu.make_async_copy(cidx_hbm.at[pl.ds(off, CK)], ci_v.at[b], isem[b]).wait()
        pltpu.async_copy(cidx_hbm.at[pl.ds(cg(i + 1), CK)], ci_v.at[1 - b], isem[1 - b])
        @pl.when(i >= 2)
        def _():
            pltpu.make_async_copy(wbuf[b], w_hbm.at[pl.ds(cg(i - 2), CK)], wsem[b]).wait()
        for j in range(CK // L):
            sl = pl.ds(j * L, L)
            idx16 = ci_v[b, sl]
            c = plsc.load_gather(p0_v, [idx16]) + plsc.load_gather(p1_v, [idx16])
            wbuf[b][sl] = 1.0 / jnp.maximum(c, 1.0)
        pltpu.async_copy(wbuf[b], w_hbm.at[pl.ds(off, CK)], wsem[b])

    pltpu.async_copy(cidx_hbm.at[pl.ds(cg(0), CK)], ci_v.at[0], isem[0])

    def pair(k, _):
        i = k * 2
        step(i, 0)
        step(i + 1, 1)
        return 0
    lax.fori_loop(0, NCT // 2, pair, 0)

    pltpu.make_async_copy(cidx_hbm.at[pl.ds(cg(NCT), CK)], ci_v.at[0], isem[0]).wait()
    pltpu.make_async_copy(wbuf[0], w_hbm.at[pl.ds(cg(NCT - 2), CK)], wsem[0]).wait()
    pltpu.make_async_copy(wbuf[1], w_hbm.at[pl.ds(cg(NCT - 1), CK)], wsem[1]).wait()

    @pl.when(wid < TAIL)
    def _():
        off = (NW * NCT + wid) * CK
        pltpu.sync_copy(cidx_hbm.at[pl.ds(off, CK)], ci_v.at[0])
        for j in range(CK // L):
            sl = pl.ds(j * L, L)
            idx16 = ci_v[0, sl]
            c = plsc.load_gather(p0_v, [idx16]) + plsc.load_gather(p1_v, [idx16])
            wbuf[0][sl] = 1.0 / jnp.maximum(c, 1.0)
        pltpu.sync_copy(wbuf[0], w_hbm.at[pl.ds(off, CK)])


@functools.partial(
    pl.kernel,
    out_type=jax.ShapeDtypeStruct((2 * N, C), jnp.float32),
    mesh=_mesh,
    compiler_params=_sc_params,
    scratch_types=[
        pltpu.VMEM((CK,), jnp.int32),
        pltpu.VMEM((CK,), jnp.int32),
        pltpu.VMEM((CK,), jnp.int32),
        pltpu.VMEM((CK,), jnp.int32),
        pltpu.VMEM((CK,), jnp.float32),
        pltpu.VMEM((CK,), jnp.float32),
        pltpu.VMEM((CK, C), jnp.float32),
        pltpu.VMEM((CK, C), jnp.float32),
        pltpu.SemaphoreType.DMA,
        pltpu.SemaphoreType.DMA,
        pltpu.SemaphoreType.DMA,
        pltpu.SemaphoreType.DMA,
        pltpu.VMEM_SHARED((N, C), jnp.float32),
    ],
)
def _agg_sc(z_hbm, gidx_hbm, dst_hbm, w_hbm, out_hbm,
            g0, g1, d0, d1, w0, w1, rows0, rows1,
            isem0, isem1, gsem0, gsem1, acc_sh):
    cid = lax.axis_index("c")
    sid = lax.axis_index("s")
    wid = sid * NC + cid
    gbuf = (g0, g1)
    dbuf = (d0, d1)
    wbuf = (w0, w1)
    rows = (rows0, rows1)
    isem = (isem0, isem1)
    gsem = (gsem0, gsem1)

    def zero_rows(i, _):
        for j in range(C // L):
            rows0[i, pl.ds(j * L, L)] = jnp.zeros((L,), jnp.float32)
        return 0
    lax.fori_loop(0, CK, zero_rows, 0)

    zb = sid * ZPT
    def zero_acc(i, _):
        pltpu.sync_copy(rows0, acc_sh.at[pl.ds(zb + i * CK, CK)])
        return 0
    lax.fori_loop(0, ZPT // CK, zero_acc, 0)
    pltpu.sync_copy(rows0.at[pl.ds(0, ZPT % CK)],
                    acc_sh.at[pl.ds(zb + (ZPT // CK) * CK, ZPT % CK)])
    @pl.when(sid == 0)
    def _():
        pltpu.sync_copy(rows0.at[pl.ds(0, N - NS * ZPT)],
                        acc_sh.at[pl.ds(NS * ZPT, N - NS * ZPT)])
    plsc.subcore_barrier()

    def cg(i):
        return (wid + jnp.minimum(i, NCT - 1) * NW) * CK

    def fire_idx(i, b):
        off = cg(i)
        pltpu.async_copy(gidx_hbm.at[pl.ds(off, CK)], gbuf[b], isem[b])
        pltpu.async_copy(dst_hbm.at[pl.ds(off, CK)], dbuf[b], isem[b])
        pltpu.async_copy(w_hbm.at[pl.ds(off, CK)], wbuf[b], isem[b])

    def wait_idx(i, b):
        off = cg(i)
        pltpu.make_async_copy(gidx_hbm.at[pl.ds(off, CK)], gbuf[b], isem[b]).wait()
        pltpu.make_async_copy(dst_hbm.at[pl.ds(off, CK)], dbuf[b], isem[b]).wait()
        pltpu.make_async_copy(w_hbm.at[pl.ds(off, CK)], wbuf[b], isem[b]).wait()

    def scale_scatter(b):
        def scale(e, _):
            w16 = plsc.load_gather(wbuf[b], [jnp.full((L,), 0, jnp.int32) + e])
            for j in range(C // L):
                sl = pl.ds(j * L, L)
                rows[b][e, sl] = rows[b][e, sl] * w16
            return 0
        lax.fori_loop(0, CK, scale, 0)
        pltpu.sync_copy(rows[b], acc_sh.at[dbuf[b]], add=True)

    fire_idx(0, 0)
    fire_idx(1, 1)
    wait_idx(0, 0)
    pltpu.async_copy(z_hbm.at[gbuf[0]], rows[0], gsem[0])

    def step(i, b):
        pltpu.make_async_copy(z_hbm.at[gbuf[b]], rows[b], gsem[b]).wait()
        wait_idx(i + 1, 1 - b)
        pltpu.async_copy(z_hbm.at[gbuf[1 - b]], rows[1 - b], gsem[1 - b])
        scale_scatter(b)
        fire_idx(i + 2, b)

    def pair(k, _):
        i = k * 2
        step(i, 0)
        step(i + 1, 1)
        return 0
    lax.fori_loop(0, NCT // 2, pair, 0)

    pltpu.make_async_copy(z_hbm.at[gbuf[0]], rows[0], gsem[0]).wait()
    wait_idx(NCT + 1, 1)

    @pl.when(wid < TAIL)
    def _():
        off = (NW * NCT + wid) * CK
        pltpu.sync_copy(gidx_hbm.at[pl.ds(off, CK)], gbuf[0])
        pltpu.sync_copy(dst_hbm.at[pl.ds(off, CK)], dbuf[0])
        pltpu.sync_copy(w_hbm.at[pl.ds(off, CK)], wbuf[0])
        pltpu.async_copy(z_hbm.at[gbuf[0]], rows[0], gsem[0]).wait()
        scale_scatter(0)

    plsc.subcore_barrier()
    ob = cid * N
    def writeout(i, _):
        pltpu.sync_copy(acc_sh.at[pl.ds(sid * ZPT + i * CK, CK)],
                        out_hbm.at[pl.ds(ob + sid * ZPT + i * CK, CK)])
        return 0
    lax.fori_loop(0, ZPT // CK, writeout, 0)
    pltpu.sync_copy(acc_sh.at[pl.ds(sid * ZPT + (ZPT // CK) * CK, ZPT % CK)],
                    out_hbm.at[pl.ds(ob + sid * ZPT + (ZPT // CK) * CK, ZPT % CK)])
    @pl.when(sid == 0)
    def _():
        pltpu.sync_copy(acc_sh.at[pl.ds(NS * ZPT, N - NS * ZPT)],
                        out_hbm.at[pl.ds(ob + NS * ZPT, N - NS * ZPT)])


def _prep_tc_body(comp0_ref, b0_ref, comp1_ref, b1_ref, g_ref, be_ref, m_ref,
                  v_ref, w0_ref, w1_ref, ab_ref):
    w0_ref[...] = jnp.dot(comp0_ref[...], b0_ref[...],
                          preferred_element_type=jnp.float32)
    w1_ref[...] = jnp.dot(comp1_ref[...], b1_ref[...],
                          preferred_element_type=jnp.float32)
    a = g_ref[...] * lax.rsqrt(v_ref[...] + EPS_BN)
    ab_ref[0:1, :] = a
    ab_ref[1:2, :] = be_ref[...] - m_ref[...] * a


_prep_tc = pl.pallas_call(
    _prep_tc_body,
    out_shape=(
        jax.ShapeDtypeStruct((R, C * C), jnp.float32),
        jax.ShapeDtypeStruct((R, C * C), jnp.float32),
        jax.ShapeDtypeStruct((2, C), jnp.float32),
    ),
)


def _proj_tc_body(h_ref, w_ref, root_ref, bias_ref, z_ref, base_ref):
    h = h_ref[...]
    for r in range(R):
        z_ref[r] = jnp.dot(h, w_ref[r], preferred_element_type=jnp.float32)
    base_ref[...] = jnp.dot(h, root_ref[...],
                            preferred_element_type=jnp.float32) + bias_ref[...]


_proj_tc = pl.pallas_call(
    _proj_tc_body,
    grid=(GRID,),
    in_specs=[
        pl.BlockSpec((NBK, C), lambda i: (i, 0)),
        pl.BlockSpec((R, C, C), lambda i: (0, 0, 0)),
        pl.BlockSpec((C, C), lambda i: (0, 0)),
        pl.BlockSpec((1, C), lambda i: (0, 0)),
    ],
    out_specs=(
        pl.BlockSpec((R, NBK, C), lambda i: (0, i, 0)),
        pl.BlockSpec((NBK, C), lambda i: (i, 0)),
    ),
    out_shape=(
        jax.ShapeDtypeStruct((R, N, C), jnp.float32),
        jax.ShapeDtypeStruct((N, C), jnp.float32),
    ),
)


def _proj2_tc_body(base0_ref, p0_ref, p1_ref, ab_ref, w_ref, root_ref,
                   bias_ref, z_ref, base_ref):
    y = base0_ref[...] + p0_ref[...] + p1_ref[...]
    h = jnp.maximum(y * ab_ref[0:1, :] + ab_ref[1:2, :], 0.0)
    for r in range(R):
        z_ref[r] = jnp.dot(h, w_ref[r], preferred_element_type=jnp.float32)
    base_ref[...] = jnp.dot(h, root_ref[...],
                            preferred_element_type=jnp.float32) + bias_ref[...]


_proj2_tc = pl.pallas_call(
    _proj2_tc_body,
    grid=(GRID,),
    in_specs=[
        pl.BlockSpec((NBK, C), lambda i: (i, 0)),
        pl.BlockSpec((NBK, C), lambda i: (i, 0)),
        pl.BlockSpec((NBK, C), lambda i: (i, 0)),
        pl.BlockSpec((2, C), lambda i: (0, 0)),
        pl.BlockSpec((R, C, C), lambda i: (0, 0, 0)),
        pl.BlockSpec((C, C), lambda i: (0, 0)),
        pl.BlockSpec((1, C), lambda i: (0, 0)),
    ],
    out_specs=(
        pl.BlockSpec((R, NBK, C), lambda i: (0, i, 0)),
        pl.BlockSpec((NBK, C), lambda i: (i, 0)),
    ),
    out_shape=(
        jax.ShapeDtypeStruct((R, N, C), jnp.float32),
        jax.ShapeDtypeStruct((N, C), jnp.float32),
    ),
)


def _final_tc_body(base_ref, p0_ref, p1_ref, out_ref):
    y = base_ref[...] + p0_ref[...] + p1_ref[...]
    nrm = jnp.sqrt(jnp.sum(y * y, axis=-1, keepdims=True))
    out_ref[...] = y / jnp.maximum(nrm, EPS_NORM)


_final_tc = pl.pallas_call(
    _final_tc_body,
    grid=(GRID,),
    in_specs=[
        pl.BlockSpec((NBK, C), lambda i: (i, 0)),
        pl.BlockSpec((NBK, C), lambda i: (i, 0)),
        pl.BlockSpec((NBK, C), lambda i: (i, 0)),
    ],
    out_specs=pl.BlockSpec((NBK, C), lambda i: (i, 0)),
    out_shape=jax.ShapeDtypeStruct((N, C), jnp.float32),
)


def kernel(x, edge_index, edge_type, comp0, bases0, root0, bias0,
           bn_gamma, bn_beta, bn_mean, bn_var, comp1, bases1, root1, bias1):
    src = edge_index[0].astype(jnp.int32)
    dst = edge_index[1].astype(jnp.int32)
    rt = edge_type.astype(jnp.int32)

    w0f, w1f, ab = _prep_tc(comp0, bases0.reshape(NB, C * C),
                            comp1, bases1.reshape(NB, C * C),
                            bn_gamma.reshape(1, C), bn_beta.reshape(1, C),
                            bn_mean.reshape(1, C), bn_var.reshape(1, C))
    w0 = w0f.reshape(R, C, C)
    w1 = w1f.reshape(R, C, C)

    cnt, gidx, cidx = _counts_sc(src, rt, dst)
    w = _weights_sc(cnt, cidx)

    z0, base0 = _proj_tc(x, w0, root0, bias0.reshape(1, C))
    parts0 = _agg_sc(z0.reshape(R * N, C), gidx, dst, w)

    z1, base1 = _proj2_tc(base0, parts0[:N], parts0[N:], ab, w1, root1,
                          bias1.reshape(1, C))
    parts1 = _agg_sc(z1.reshape(R * N, C), gidx, dst, w)

    return _final_tc(base1, parts1[:N], parts1[N:])


# speedup vs baseline: 20.7369x; 1.1525x over previous
import functools

import jax
import jax.numpy as jnp
from jax import lax
from jax.experimental import pallas as pl
from jax.experimental.pallas import tpu as pltpu
from jax.experimental.pallas import tpu_sc as plsc

N = 10000
E = 320000
C = 128
R = 5
NB = 4
EPS_BN = 1e-5
EPS_NORM = 1e-12

NC = 2
NS = 16
L = 16
NW = NC * NS
CK = 128
NCHG = E // CK
NCT = NCHG // NW
TAIL = NCHG - NW * NCT
CPAD = 51200
CPT = CPAD // NS
ZPT = 624
NBK = 1000
GRID = N // NBK

_mesh = plsc.VectorSubcoreMesh(core_axis_name="c", subcore_axis_name="s")
_sc_params = pltpu.CompilerParams(needs_layout_passes=False)


@functools.partial(
    pl.kernel,
    out_type=(
        jax.ShapeDtypeStruct((2 * CPAD,), jnp.float32),
        jax.ShapeDtypeStruct((E,), jnp.int32),
        jax.ShapeDtypeStruct((E,), jnp.int32),
    ),
    mesh=_mesh,
    compiler_params=_sc_params,
    scratch_types=[
        pltpu.VMEM((2, CK), jnp.int32),
        pltpu.VMEM((2, CK), jnp.int32),
        pltpu.VMEM((2, CK), jnp.int32),
        pltpu.VMEM((CK,), jnp.int32),
        pltpu.VMEM((CK,), jnp.int32),
        pltpu.VMEM((CK,), jnp.int32),
        pltpu.VMEM((CK,), jnp.int32),
        pltpu.VMEM((CK,), jnp.float32),
        pltpu.VMEM((CPT,), jnp.float32),
        pltpu.SemaphoreType.DMA,
        pltpu.SemaphoreType.DMA,
        pltpu.SemaphoreType.DMA,
        pltpu.SemaphoreType.DMA,
        pltpu.VMEM_SHARED((CPAD,), jnp.float32),
    ],
)
def _counts_sc(src_hbm, rt_hbm, dst_hbm, cnt_hbm, gidx_hbm, cidx_hbm,
               src_v, rt_v, dst_v, g0, g1, c0, c1, ones_v, zer_v,
               isem0, isem1, wsem0, wsem1, acc_sh):
    cid = lax.axis_index("c")
    sid = lax.axis_index("s")
    wid = sid * NC + cid
    gbuf = (g0, g1)
    cbuf = (c0, c1)
    isem = (isem0, isem1)
    wsem = (wsem0, wsem1)

    def fill_ones(i, _):
        ones_v[pl.ds(i * L, L)] = jnp.full((L,), 1.0, jnp.float32)
        return 0
    lax.fori_loop(0, CK // L, fill_ones, 0)

    def fill_zeros(i, _):
        zer_v[pl.ds(i * L, L)] = jnp.zeros((L,), jnp.float32)
        return 0
    lax.fori_loop(0, CPT // L, fill_zeros, 0)

    pltpu.sync_copy(zer_v, acc_sh.at[pl.ds(sid * CPT, CPT)])
    plsc.subcore_barrier()

    def cg(i):
        return (wid + jnp.minimum(i, NCT - 1) * NW) * CK

    def fire_idx(i, b):
        off = cg(i)
        pltpu.async_copy(src_hbm.at[pl.ds(off, CK)], src_v.at[b], isem[b])
        pltpu.async_copy(rt_hbm.at[pl.ds(off, CK)], rt_v.at[b], isem[b])
        pltpu.async_copy(dst_hbm.at[pl.ds(off, CK)], dst_v.at[b], isem[b])

    def wait_idx(i, b):
        off = cg(i)
        pltpu.make_async_copy(src_hbm.at[pl.ds(off, CK)], src_v.at[b], isem[b]).wait()
        pltpu.make_async_copy(rt_hbm.at[pl.ds(off, CK)], rt_v.at[b], isem[b]).wait()
        pltpu.make_async_copy(dst_hbm.at[pl.ds(off, CK)], dst_v.at[b], isem[b]).wait()

    def wait_writes(i, b):
        off = cg(i)
        pltpu.make_async_copy(gbuf[b], gidx_hbm.at[pl.ds(off, CK)], wsem[b]).wait()
        pltpu.make_async_copy(cbuf[b], cidx_hbm.at[pl.ds(off, CK)], wsem[b]).wait()

    def compute(b):
        for j in range(CK // L):
            sl = pl.ds(j * L, L)
            s16 = src_v[b, sl]
            r16 = rt_v[b, sl]
            d16 = dst_v[b, sl]
            gbuf[b][sl] = r16 * N + s16
            cbuf[b][sl] = d16 * R + r16

    def step(i, b, first):
        wait_idx(i, b)
        fire_idx(i + 1, 1 - b)
        if first:
            pass
        else:
            @pl.when(i >= 2)
            def _():
                wait_writes(i - 2, b)
        compute(b)
        off = cg(i)
        pltpu.async_copy(gbuf[b], gidx_hbm.at[pl.ds(off, CK)], wsem[b])
        pltpu.async_copy(cbuf[b], cidx_hbm.at[pl.ds(off, CK)], wsem[b])
        pltpu.sync_copy(ones_v, acc_sh.at[cbuf[b]], add=True)

    fire_idx(0, 0)

    def pair(k, _):
        i = k * 2
        step(i, 0, False)
        step(i + 1, 1, False)
        return 0
    lax.fori_loop(0, NCT // 2, pair, 0)

    wait_idx(NCT, 0)
    wait_writes(NCT - 2, 0)
    wait_writes(NCT - 1, 1)

    @pl.when(wid < TAIL)
    def _():
        off = (NW * NCT + wid) * CK
        pltpu.sync_copy(src_hbm.at[pl.ds(off, CK)], src_v.at[0])
        pltpu.sync_copy(rt_hbm.at[pl.ds(off, CK)], rt_v.at[0])
        pltpu.sync_copy(dst_hbm.at[pl.ds(off, CK)], dst_v.at[0])
        compute(0)
        pltpu.sync_copy(gbuf[0], gidx_hbm.at[pl.ds(off, CK)])
        pltpu.sync_copy(cbuf[0], cidx_hbm.at[pl.ds(off, CK)])
        pltpu.sync_copy(ones_v, acc_sh.at[cbuf[0]], add=True)

    plsc.subcore_barrier()
    pltpu.sync_copy(acc_sh.at[pl.ds(sid * CPT, CPT)],
                    cnt_hbm.at[pl.ds(cid * CPAD + sid * CPT, CPT)])


@functools.partial(
    pl.kernel,
    out_type=jax.ShapeDtypeStruct((E,), jnp.float32),
    mesh=_mesh,
    compiler_params=_sc_params,
    scratch_types=[
        pltpu.VMEM((CPAD,), jnp.float32),
        pltpu.VMEM((CPAD,), jnp.float32),
        pltpu.VMEM((2, CK), jnp.int32),
        pltpu.VMEM((CK,), jnp.float32),
        pltpu.VMEM((CK,), jnp.float32),
        pltpu.SemaphoreType.DMA,
        pltpu.SemaphoreType.DMA,
        pltpu.SemaphoreType.DMA,
        pltpu.SemaphoreType.DMA,
    ],
)
def _weights_sc(cnt_hbm, cidx_hbm, w_hbm, p0_v, p1_v, ci_v, w0, w1,
                isem0, isem1, wsem0, wsem1):
    wid = lax.axis_index("s") * NC + lax.axis_index("c")
    wbuf = (w0, w1)
    isem = (isem0, isem1)
    wsem = (wsem0, wsem1)
    pltpu.sync_copy(cnt_hbm.at[pl.ds(0, CPAD)], p0_v)
    pltpu.sync_copy(cnt_hbm.at[pl.ds(CPAD, CPAD)], p1_v)

    def cg(i):
        return (wid + jnp.minimum(i, NCT - 1) * NW) * CK

    def step(i, b):
        off = cg(i)
        pltpu.make_async_copy(cidx_hbm.at[pl.ds(off, CK)], ci_v.at[b], isem[b]).wait()
        pltpu.async_copy(cidx_hbm.at[pl.ds(cg(i + 1), CK)], ci_v.at[1 - b], isem[1 - b])
        @pl.when(i >= 2)
        def _():
            pltpu.make_async_copy(wbuf[b], w_hbm.at[pl.ds(cg(i - 2), CK)], wsem[b]).wait()
        for j in range(CK // L):
            sl = pl.ds(j * L, L)
            idx16 = ci_v[b, sl]
            c = plsc.load_gather(p0_v, [idx16]) + plsc.load_gather(p1_v, [idx16])
            wbuf[b][sl] = 1.0 / jnp.maximum(c, 1.0)
        pltpu.async_copy(wbuf[b], w_hbm.at[pl.ds(off, CK)], wsem[b])

    pltpu.async_copy(cidx_hbm.at[pl.ds(cg(0), CK)], ci_v.at[0], isem[0])

    def pair(k, _):
        i = k * 2
        step(i, 0)
        step(i + 1, 1)
        return 0
    lax.fori_loop(0, NCT // 2, pair, 0)

    pltpu.make_async_copy(cidx_hbm.at[pl.ds(cg(NCT), CK)], ci_v.at[0], isem[0]).wait()
    pltpu.make_async_copy(wbuf[0], w_hbm.at[pl.ds(cg(NCT - 2), CK)], wsem[0]).wait()
    pltpu.make_async_copy(wbuf[1], w_hbm.at[pl.ds(cg(NCT - 1), CK)], wsem[1]).wait()

    @pl.when(wid < TAIL)
    def _():
        off = (NW * NCT + wid) * CK
        pltpu.sync_copy(cidx_hbm.at[pl.ds(off, CK)], ci_v.at[0])
        for j in range(CK // L):
            sl = pl.ds(j * L, L)
            idx16 = ci_v[0, sl]
            c = plsc.load_gather(p0_v, [idx16]) + plsc.load_gather(p1_v, [idx16])
            wbuf[0][sl] = 1.0 / jnp.maximum(c, 1.0)
        pltpu.sync_copy(wbuf[0], w_hbm.at[pl.ds(off, CK)])


@functools.partial(
    pl.kernel,
    out_type=jax.ShapeDtypeStruct((2 * N, C), jnp.float32),
    mesh=_mesh,
    compiler_params=_sc_params,
    scratch_types=[
        pltpu.VMEM((CK,), jnp.int32),
        pltpu.VMEM((CK,), jnp.int32),
        pltpu.VMEM((CK,), jnp.int32),
        pltpu.VMEM((CK,), jnp.int32),
        pltpu.VMEM((CK,), jnp.int32),
        pltpu.VMEM((CK,), jnp.int32),
        pltpu.VMEM((CK,), jnp.float32),
        pltpu.VMEM((CK,), jnp.float32),
        pltpu.VMEM((CK,), jnp.float32),
        pltpu.VMEM((CK, C), jnp.float32),
        pltpu.VMEM((CK, C), jnp.float32),
        pltpu.SemaphoreType.DMA,
        pltpu.SemaphoreType.DMA,
        pltpu.SemaphoreType.DMA,
        pltpu.SemaphoreType.DMA,
        pltpu.SemaphoreType.DMA,
        pltpu.SemaphoreType.DMA,
        pltpu.SemaphoreType.DMA,
        pltpu.VMEM_SHARED((N, C), jnp.float32),
    ],
)
def _agg_sc(z_hbm, gidx_hbm, dst_hbm, w_hbm, out_hbm,
            g0, g1, g2, d0, d1, d2, w0, w1, w2, rows0, rows1,
            isem0, isem1, isem2, gsem0, gsem1, ssem0, ssem1, acc_sh):
    cid = lax.axis_index("c")
    sid = lax.axis_index("s")
    wid = sid * NC + cid
    gbuf = (g0, g1, g2)
    dbuf = (d0, d1, d2)
    wbuf = (w0, w1, w2)
    rows = (rows0, rows1)
    isem = (isem0, isem1, isem2)
    gsem = (gsem0, gsem1)
    ssem = (ssem0, ssem1)

    def zero_rows(i, _):
        for j in range(C // L):
            rows0[i, pl.ds(j * L, L)] = jnp.zeros((L,), jnp.float32)
        return 0
    lax.fori_loop(0, CK, zero_rows, 0)

    zb = sid * ZPT
    def zero_acc(i, _):
        pltpu.sync_copy(rows0, acc_sh.at[pl.ds(zb + i * CK, CK)])
        return 0
    lax.fori_loop(0, ZPT // CK, zero_acc, 0)
    pltpu.sync_copy(rows0.at[pl.ds(0, ZPT % CK)],
                    acc_sh.at[pl.ds(zb + (ZPT // CK) * CK, ZPT % CK)])
    @pl.when(sid == 0)
    def _():
        pltpu.sync_copy(rows0.at[pl.ds(0, N - NS * ZPT)],
                        acc_sh.at[pl.ds(NS * ZPT, N - NS * ZPT)])
    plsc.subcore_barrier()

    def cg(i):
        return (wid + jnp.minimum(i, NCT - 1) * NW) * CK

    def fire_idx(i, t):
        off = cg(i)
        pltpu.async_copy(gidx_hbm.at[pl.ds(off, CK)], gbuf[t], isem[t])
        pltpu.async_copy(dst_hbm.at[pl.ds(off, CK)], dbuf[t], isem[t])
        pltpu.async_copy(w_hbm.at[pl.ds(off, CK)], wbuf[t], isem[t])

    def wait_idx(i, t):
        off = cg(i)
        pltpu.make_async_copy(gidx_hbm.at[pl.ds(off, CK)], gbuf[t], isem[t]).wait()
        pltpu.make_async_copy(dst_hbm.at[pl.ds(off, CK)], dbuf[t], isem[t]).wait()
        pltpu.make_async_copy(w_hbm.at[pl.ds(off, CK)], wbuf[t], isem[t]).wait()

    def scale(b, t):
        def body(e, _):
            w16 = plsc.load_gather(wbuf[t], [jnp.full((L,), 0, jnp.int32) + e])
            for j in range(C // L):
                sl = pl.ds(j * L, L)
                rows[b][e, sl] = rows[b][e, sl] * w16
            return 0
        lax.fori_loop(0, CK, body, 0)

    fire_idx(0, 0)
    fire_idx(1, 1)
    wait_idx(0, 0)
    pltpu.async_copy(z_hbm.at[gbuf[0]], rows[0], gsem[0])

    def step(i, k, b, t, u):
        t1 = (t + 1) % 3
        t2 = (t + 2) % 3
        pltpu.make_async_copy(z_hbm.at[gbuf[t]], rows[b], gsem[b]).wait()
        def wait_prev_scatter():
            pltpu.make_async_copy(rows[1 - b], acc_sh.at[dbuf[t2]],
                                  ssem[1 - b]).wait()
        if u == 0:
            @pl.when(k > 0)
            def _():
                wait_prev_scatter()
        else:
            wait_prev_scatter()
        wait_idx(i + 1, t1)
        pltpu.async_copy(z_hbm.at[gbuf[t1]], rows[1 - b], gsem[1 - b])
        scale(b, t)
        pltpu.async_copy(rows[b], acc_sh.at[dbuf[t]], ssem[b], add=True)
        fire_idx(i + 2, t2)

    def block(k, _):
        for u in range(6):
            step(k * 6 + u, k, u % 2, u % 3, u)
        return 0
    lax.fori_loop(0, NCT // 6, block, 0)

    pltpu.make_async_copy(z_hbm.at[gbuf[NCT % 3]], rows[0], gsem[0]).wait()
    pltpu.make_async_copy(rows[1], acc_sh.at[dbuf[(NCT - 1) % 3]], ssem[1]).wait()
    wait_idx(NCT + 1, (NCT + 1) % 3)

    @pl.when(wid < TAIL)
    def _():
        off = (NW * NCT + wid) * CK
        pltpu.sync_copy(gidx_hbm.at[pl.ds(off, CK)], gbuf[0])
        pltpu.sync_copy(dst_hbm.at[pl.ds(off, CK)], dbuf[0])
        pltpu.sync_copy(w_hbm.at[pl.ds(off, CK)], wbuf[0])
        pltpu.async_copy(z_hbm.at[gbuf[0]], rows[0], gsem[0]).wait()
        scale(0, 0)
        pltpu.sync_copy(rows[0], acc_sh.at[dbuf[0]], add=True)

    plsc.subcore_barrier()
    ob = cid * N
    def writeout(i, _):
        pltpu.sync_copy(acc_sh.at[pl.ds(sid * ZPT + i * CK, CK)],
                        out_hbm.at[pl.ds(ob + sid * ZPT + i * CK, CK)])
        return 0
    lax.fori_loop(0, ZPT // CK, writeout, 0)
    pltpu.sync_copy(acc_sh.at[pl.ds(sid * ZPT + (ZPT // CK) * CK, ZPT % CK)],
                    out_hbm.at[pl.ds(ob + sid * ZPT + (ZPT // CK) * CK, ZPT % CK)])
    @pl.when(sid == 0)
    def _():
        pltpu.sync_copy(acc_sh.at[pl.ds(NS * ZPT, N - NS * ZPT)],
                        out_hbm.at[pl.ds(ob + NS * ZPT, N - NS * ZPT)])


def _prep_tc_body(comp0_ref, b0_ref, comp1_ref, b1_ref, g_ref, be_ref, m_ref,
                  v_ref, w0_ref, w1_ref, ab_ref):
    w0_ref[...] = jnp.dot(comp0_ref[...], b0_ref[...],
                          preferred_element_type=jnp.float32)
    w1_ref[...] = jnp.dot(comp1_ref[...], b1_ref[...],
                          preferred_element_type=jnp.float32)
    a = g_ref[...] * lax.rsqrt(v_ref[...] + EPS_BN)
    ab_ref[0:1, :] = a
    ab_ref[1:2, :] = be_ref[...] - m_ref[...] * a


_prep_tc = pl.pallas_call(
    _prep_tc_body,
    out_shape=(
        jax.ShapeDtypeStruct((R, C * C), jnp.float32),
        jax.ShapeDtypeStruct((R, C * C), jnp.float32),
        jax.ShapeDtypeStruct((2, C), jnp.float32),
    ),
)


def _proj_tc_body(h_ref, w_ref, root_ref, bias_ref, z_ref, base_ref):
    h = h_ref[...]
    for r in range(R):
        z_ref[r] = jnp.dot(h, w_ref[r], preferred_element_type=jnp.float32)
    base_ref[...] = jnp.dot(h, root_ref[...],
                            preferred_element_type=jnp.float32) + bias_ref[...]


_proj_tc = pl.pallas_call(
    _proj_tc_body,
    grid=(GRID,),
    in_specs=[
        pl.BlockSpec((NBK, C), lambda i: (i, 0)),
        pl.BlockSpec((R, C, C), lambda i: (0, 0, 0)),
        pl.BlockSpec((C, C), lambda i: (0, 0)),
        pl.BlockSpec((1, C), lambda i: (0, 0)),
    ],
    out_specs=(
        pl.BlockSpec((R, NBK, C), lambda i: (0, i, 0)),
        pl.BlockSpec((NBK, C), lambda i: (i, 0)),
    ),
    out_shape=(
        jax.ShapeDtypeStruct((R, N, C), jnp.float32),
        jax.ShapeDtypeStruct((N, C), jnp.float32),
    ),
)


def _proj2_tc_body(base0_ref, p0_ref, p1_ref, ab_ref, w_ref, root_ref,
                   bias_ref, z_ref, base_ref):
    y = base0_ref[...] + p0_ref[...] + p1_ref[...]
    h = jnp.maximum(y * ab_ref[0:1, :] + ab_ref[1:2, :], 0.0)
    for r in range(R):
        z_ref[r] = jnp.dot(h, w_ref[r], preferred_element_type=jnp.float32)
    base_ref[...] = jnp.dot(h, root_ref[...],
                            preferred_element_type=jnp.float32) + bias_ref[...]


_proj2_tc = pl.pallas_call(
    _proj2_tc_body,
    grid=(GRID,),
    in_specs=[
        pl.BlockSpec((NBK, C), lambda i: (i, 0)),
        pl.BlockSpec((NBK, C), lambda i: (i, 0)),
        pl.BlockSpec((NBK, C), lambda i: (i, 0)),
        pl.BlockSpec((2, C), lambda i: (0, 0)),
        pl.BlockSpec((R, C, C), lambda i: (0, 0, 0)),
        pl.BlockSpec((C, C), lambda i: (0, 0)),
        pl.BlockSpec((1, C), lambda i: (0, 0)),
    ],
    out_specs=(
        pl.BlockSpec((R, NBK, C), lambda i: (0, i, 0)),
        pl.BlockSpec((NBK, C), lambda i: (i, 0)),
    ),
    out_shape=(
        jax.ShapeDtypeStruct((R, N, C), jnp.float32),
        jax.ShapeDtypeStruct((N, C), jnp.float32),
    ),
)


def _final_tc_body(base_ref, p0_ref, p1_ref, out_ref):
    y = base_ref[...] + p0_ref[...] + p1_ref[...]
    nrm = jnp.sqrt(jnp.sum(y * y, axis=-1, keepdims=True))
    out_ref[...] = y / jnp.maximum(nrm, EPS_NORM)


_final_tc = pl.pallas_call(
    _final_tc_body,
    grid=(GRID,),
    in_specs=[
        pl.BlockSpec((NBK, C), lambda i: (i, 0)),
        pl.BlockSpec((NBK, C), lambda i: (i, 0)),
        pl.BlockSpec((NBK, C), lambda i: (i, 0)),
    ],
    out_specs=pl.BlockSpec((NBK, C), lambda i: (i, 0)),
    out_shape=jax.ShapeDtypeStruct((N, C), jnp.float32),
)


def kernel(x, edge_index, edge_type, comp0, bases0, root0, bias0,
           bn_gamma, bn_beta, bn_mean, bn_var, comp1, bases1, root1, bias1):
    src = edge_index[0].astype(jnp.int32)
    dst = edge_index[1].astype(jnp.int32)
    rt = edge_type.astype(jnp.int32)

    w0f, w1f, ab = _prep_tc(comp0, bases0.reshape(NB, C * C),
                            comp1, bases1.reshape(NB, C * C),
                            bn_gamma.reshape(1, C), bn_beta.reshape(1, C),
                            bn_mean.reshape(1, C), bn_var.reshape(1, C))
    w0 = w0f.reshape(R, C, C)
    w1 = w1f.reshape(R, C, C)

    cnt, gidx, cidx = _counts_sc(src, rt, dst)
    w = _weights_sc(cnt, cidx)

    z0, base0 = _proj_tc(x, w0, root0, bias0.reshape(1, C))
    parts0 = _agg_sc(z0.reshape(R * N, C), gidx, dst, w)

    z1, base1 = _proj2_tc(base0, parts0[:N], parts0[N:], ab, w1, root1,
                          bias1.reshape(1, C))
    parts1 = _agg_sc(z1.reshape(R * N, C), gidx, dst, w)

    return _final_tc(base1, parts1[:N], parts1[N:])


# speedup vs baseline: 23.4220x; 1.1295x over previous
import functools

import jax
import jax.numpy as jnp
from jax import lax
from jax.experimental import pallas as pl
from jax.experimental.pallas import tpu as pltpu
from jax.experimental.pallas import tpu_sc as plsc

N = 10000
E = 320000
C = 128
R = 5
NB = 4
EPS_BN = 1e-5
EPS_NORM = 1e-12

NC = 2
NS = 16
L = 16
NW = NC * NS
CK = 128
NCHG = E // CK
NCT = NCHG // NW
TAIL = NCHG - NW * NCT
CPAD = 51200
CPT = CPAD // NS
ZPT = 624
NBK = 1000
GRID = N // NBK

_mesh = plsc.VectorSubcoreMesh(core_axis_name="c", subcore_axis_name="s")
_sc_params = pltpu.CompilerParams(needs_layout_passes=False)


@functools.partial(
    pl.kernel,
    out_type=(
        jax.ShapeDtypeStruct((2 * CPAD,), jnp.float32),
        jax.ShapeDtypeStruct((E,), jnp.int32),
        jax.ShapeDtypeStruct((E,), jnp.int32),
    ),
    mesh=_mesh,
    compiler_params=_sc_params,
    scratch_types=[
        pltpu.VMEM((2, CK), jnp.int32),
        pltpu.VMEM((2, CK), jnp.int32),
        pltpu.VMEM((2, CK), jnp.int32),
        pltpu.VMEM((CK,), jnp.int32),
        pltpu.VMEM((CK,), jnp.int32),
        pltpu.VMEM((CK,), jnp.int32),
        pltpu.VMEM((CK,), jnp.int32),
        pltpu.VMEM((CK,), jnp.float32),
        pltpu.VMEM((CPT,), jnp.float32),
        pltpu.SemaphoreType.DMA,
        pltpu.SemaphoreType.DMA,
        pltpu.SemaphoreType.DMA,
        pltpu.SemaphoreType.DMA,
        pltpu.VMEM_SHARED((CPAD,), jnp.float32),
    ],
)
def _counts_sc(src_hbm, rt_hbm, dst_hbm, cnt_hbm, gidx_hbm, cidx_hbm,
               src_v, rt_v, dst_v, g0, g1, c0, c1, ones_v, zer_v,
               isem0, isem1, wsem0, wsem1, acc_sh):
    cid = lax.axis_index("c")
    sid = lax.axis_index("s")
    wid = sid * NC + cid
    gbuf = (g0, g1)
    cbuf = (c0, c1)
    isem = (isem0, isem1)
    wsem = (wsem0, wsem1)

    def fill_ones(i, _):
        ones_v[pl.ds(i * L, L)] = jnp.full((L,), 1.0, jnp.float32)
        return 0
    lax.fori_loop(0, CK // L, fill_ones, 0)

    def fill_zeros(i, _):
        zer_v[pl.ds(i * L, L)] = jnp.zeros((L,), jnp.float32)
        return 0
    lax.fori_loop(0, CPT // L, fill_zeros, 0)

    pltpu.sync_copy(zer_v, acc_sh.at[pl.ds(sid * CPT, CPT)])
    plsc.subcore_barrier()

    def cg(i):
        return (wid + jnp.minimum(i, NCT - 1) * NW) * CK

    def fire_idx(i, b):
        off = cg(i)
        pltpu.async_copy(src_hbm.at[pl.ds(off, CK)], src_v.at[b], isem[b])
        pltpu.async_copy(rt_hbm.at[pl.ds(off, CK)], rt_v.at[b], isem[b])
        pltpu.async_copy(dst_hbm.at[pl.ds(off, CK)], dst_v.at[b], isem[b])

    def wait_idx(i, b):
        off = cg(i)
        pltpu.make_async_copy(src_hbm.at[pl.ds(off, CK)], src_v.at[b], isem[b]).wait()
        pltpu.make_async_copy(rt_hbm.at[pl.ds(off, CK)], rt_v.at[b], isem[b]).wait()
        pltpu.make_async_copy(dst_hbm.at[pl.ds(off, CK)], dst_v.at[b], isem[b]).wait()

    def wait_writes(i, b):
        off = cg(i)
        pltpu.make_async_copy(gbuf[b], gidx_hbm.at[pl.ds(off, CK)], wsem[b]).wait()
        pltpu.make_async_copy(cbuf[b], cidx_hbm.at[pl.ds(off, CK)], wsem[b]).wait()

    def compute(b):
        for j in range(CK // L):
            sl = pl.ds(j * L, L)
            s16 = src_v[b, sl]
            r16 = rt_v[b, sl]
            d16 = dst_v[b, sl]
            gbuf[b][sl] = r16 * N + s16
            cbuf[b][sl] = d16 * R + r16

    def step(i, b, first):
        wait_idx(i, b)
        fire_idx(i + 1, 1 - b)
        if first:
            pass
        else:
            @pl.when(i >= 2)
            def _():
                wait_writes(i - 2, b)
        compute(b)
        off = cg(i)
        pltpu.async_copy(gbuf[b], gidx_hbm.at[pl.ds(off, CK)], wsem[b])
        pltpu.async_copy(cbuf[b], cidx_hbm.at[pl.ds(off, CK)], wsem[b])
        pltpu.sync_copy(ones_v, acc_sh.at[cbuf[b]], add=True)

    fire_idx(0, 0)

    def pair(k, _):
        i = k * 2
        step(i, 0, False)
        step(i + 1, 1, False)
        return 0
    lax.fori_loop(0, NCT // 2, pair, 0)

    wait_idx(NCT, 0)
    wait_writes(NCT - 2, 0)
    wait_writes(NCT - 1, 1)

    @pl.when(wid < TAIL)
    def _():
        off = (NW * NCT + wid) * CK
        pltpu.sync_copy(src_hbm.at[pl.ds(off, CK)], src_v.at[0])
        pltpu.sync_copy(rt_hbm.at[pl.ds(off, CK)], rt_v.at[0])
        pltpu.sync_copy(dst_hbm.at[pl.ds(off, CK)], dst_v.at[0])
        compute(0)
        pltpu.sync_copy(gbuf[0], gidx_hbm.at[pl.ds(off, CK)])
        pltpu.sync_copy(cbuf[0], cidx_hbm.at[pl.ds(off, CK)])
        pltpu.sync_copy(ones_v, acc_sh.at[cbuf[0]], add=True)

    plsc.subcore_barrier()
    pltpu.sync_copy(acc_sh.at[pl.ds(sid * CPT, CPT)],
                    cnt_hbm.at[pl.ds(cid * CPAD + sid * CPT, CPT)])


def _rc_tc_body(cnt_ref, rc_ref):
    c = cnt_ref[0:CPAD // C, :] + cnt_ref[CPAD // C:, :]
    rc_ref[...] = 1.0 / jnp.maximum(c, 1.0)


_rc_tc = pl.pallas_call(
    _rc_tc_body,
    out_shape=jax.ShapeDtypeStruct((CPAD // C, C), jnp.float32),
)


@functools.partial(
    pl.kernel,
    out_type=jax.ShapeDtypeStruct((E,), jnp.float32),
    mesh=_mesh,
    compiler_params=_sc_params,
    scratch_types=[
        pltpu.VMEM((CPAD,), jnp.float32),
        pltpu.VMEM((2, CK), jnp.int32),
        pltpu.VMEM((CK,), jnp.float32),
        pltpu.VMEM((CK,), jnp.float32),
        pltpu.SemaphoreType.DMA,
        pltpu.SemaphoreType.DMA,
        pltpu.SemaphoreType.DMA,
        pltpu.SemaphoreType.DMA,
    ],
)
def _weights_sc(rc_hbm, cidx_hbm, w_hbm, rc_v, ci_v, w0, w1,
                isem0, isem1, wsem0, wsem1):
    wid = lax.axis_index("s") * NC + lax.axis_index("c")
    wbuf = (w0, w1)
    isem = (isem0, isem1)
    wsem = (wsem0, wsem1)
    pltpu.sync_copy(rc_hbm, rc_v)

    def cg(i):
        return (wid + jnp.minimum(i, NCT - 1) * NW) * CK

    def step(i, b):
        off = cg(i)
        pltpu.make_async_copy(cidx_hbm.at[pl.ds(off, CK)], ci_v.at[b], isem[b]).wait()
        pltpu.async_copy(cidx_hbm.at[pl.ds(cg(i + 1), CK)], ci_v.at[1 - b], isem[1 - b])
        @pl.when(i >= 2)
        def _():
            pltpu.make_async_copy(wbuf[b], w_hbm.at[pl.ds(cg(i - 2), CK)], wsem[b]).wait()
        for j in range(CK // L):
            sl = pl.ds(j * L, L)
            wbuf[b][sl] = plsc.load_gather(rc_v, [ci_v[b, sl]])
        pltpu.async_copy(wbuf[b], w_hbm.at[pl.ds(off, CK)], wsem[b])

    pltpu.async_copy(cidx_hbm.at[pl.ds(cg(0), CK)], ci_v.at[0], isem[0])

    def pair(k, _):
        i = k * 2
        step(i, 0)
        step(i + 1, 1)
        return 0
    lax.fori_loop(0, NCT // 2, pair, 0)

    pltpu.make_async_copy(cidx_hbm.at[pl.ds(cg(NCT), CK)], ci_v.at[0], isem[0]).wait()
    pltpu.make_async_copy(wbuf[0], w_hbm.at[pl.ds(cg(NCT - 2), CK)], wsem[0]).wait()
    pltpu.make_async_copy(wbuf[1], w_hbm.at[pl.ds(cg(NCT - 1), CK)], wsem[1]).wait()

    @pl.when(wid < TAIL)
    def _():
        off = (NW * NCT + wid) * CK
        pltpu.sync_copy(cidx_hbm.at[pl.ds(off, CK)], ci_v.at[0])
        for j in range(CK // L):
            sl = pl.ds(j * L, L)
            wbuf[0][sl] = plsc.load_gather(rc_v, [ci_v[0, sl]])
        pltpu.sync_copy(wbuf[0], w_hbm.at[pl.ds(off, CK)])


@functools.partial(
    pl.kernel,
    out_type=jax.ShapeDtypeStruct((2 * N, C), jnp.float32),
    mesh=_mesh,
    compiler_params=_sc_params,
    scratch_types=[
        pltpu.VMEM((CK,), jnp.int32),
        pltpu.VMEM((CK,), jnp.int32),
        pltpu.VMEM((CK,), jnp.int32),
        pltpu.VMEM((CK,), jnp.int32),
        pltpu.VMEM((CK,), jnp.int32),
        pltpu.VMEM((CK,), jnp.int32),
        pltpu.VMEM((CK,), jnp.float32),
        pltpu.VMEM((CK,), jnp.float32),
        pltpu.VMEM((CK,), jnp.float32),
        pltpu.VMEM((CK, C), jnp.float32),
        pltpu.VMEM((CK, C), jnp.float32),
        pltpu.SemaphoreType.DMA,
        pltpu.SemaphoreType.DMA,
        pltpu.SemaphoreType.DMA,
        pltpu.SemaphoreType.DMA,
        pltpu.SemaphoreType.DMA,
        pltpu.SemaphoreType.DMA,
        pltpu.SemaphoreType.DMA,
        pltpu.VMEM_SHARED((N, C), jnp.float32),
    ],
)
def _agg_sc(z_hbm, gidx_hbm, dst_hbm, w_hbm, out_hbm,
            g0, g1, g2, d0, d1, d2, w0, w1, w2, rows0, rows1,
            isem0, isem1, isem2, gsem0, gsem1, ssem0, ssem1, acc_sh):
    cid = lax.axis_index("c")
    sid = lax.axis_index("s")
    wid = sid * NC + cid
    gbuf = (g0, g1, g2)
    dbuf = (d0, d1, d2)
    wbuf = (w0, w1, w2)
    rows = (rows0, rows1)
    isem = (isem0, isem1, isem2)
    gsem = (gsem0, gsem1)
    ssem = (ssem0, ssem1)

    def zero_rows(i, _):
        for j in range(C // L):
            rows0[i, pl.ds(j * L, L)] = jnp.zeros((L,), jnp.float32)
        return 0
    lax.fori_loop(0, CK, zero_rows, 0)

    zb = sid * ZPT
    def zero_acc(i, _):
        pltpu.sync_copy(rows0, acc_sh.at[pl.ds(zb + i * CK, CK)])
        return 0
    lax.fori_loop(0, ZPT // CK, zero_acc, 0)
    pltpu.sync_copy(rows0.at[pl.ds(0, ZPT % CK)],
                    acc_sh.at[pl.ds(zb + (ZPT // CK) * CK, ZPT % CK)])
    @pl.when(sid == 0)
    def _():
        pltpu.sync_copy(rows0.at[pl.ds(0, N - NS * ZPT)],
                        acc_sh.at[pl.ds(NS * ZPT, N - NS * ZPT)])
    plsc.subcore_barrier()

    def cg(i):
        return (wid + jnp.minimum(i, NCT - 1) * NW) * CK

    def fire_idx(i, t):
        off = cg(i)
        pltpu.async_copy(gidx_hbm.at[pl.ds(off, CK)], gbuf[t], isem[t])
        pltpu.async_copy(dst_hbm.at[pl.ds(off, CK)], dbuf[t], isem[t])
        pltpu.async_copy(w_hbm.at[pl.ds(off, CK)], wbuf[t], isem[t])

    def wait_idx(i, t):
        off = cg(i)
        pltpu.make_async_copy(gidx_hbm.at[pl.ds(off, CK)], gbuf[t], isem[t]).wait()
        pltpu.make_async_copy(dst_hbm.at[pl.ds(off, CK)], dbuf[t], isem[t]).wait()
        pltpu.make_async_copy(w_hbm.at[pl.ds(off, CK)], wbuf[t], isem[t]).wait()

    def scale(b, t):
        def body(h, _):
            e0 = h * 2
            e1 = h * 2 + 1
            wa = plsc.load_gather(wbuf[t], [jnp.full((L,), 0, jnp.int32) + e0])
            wb = plsc.load_gather(wbuf[t], [jnp.full((L,), 0, jnp.int32) + e1])
            for j in range(C // L):
                sl = pl.ds(j * L, L)
                rows[b][e0, sl] = rows[b][e0, sl] * wa
                rows[b][e1, sl] = rows[b][e1, sl] * wb
            return 0
        lax.fori_loop(0, CK // 2, body, 0)

    fire_idx(0, 0)
    fire_idx(1, 1)
    wait_idx(0, 0)
    pltpu.async_copy(z_hbm.at[gbuf[0]], rows[0], gsem[0])

    def step(i, k, b, t, u):
        t1 = (t + 1) % 3
        t2 = (t + 2) % 3
        pltpu.make_async_copy(z_hbm.at[gbuf[t]], rows[b], gsem[b]).wait()
        def wait_prev_scatter():
            pltpu.make_async_copy(rows[1 - b], acc_sh.at[dbuf[t2]],
                                  ssem[1 - b]).wait()
        if u == 0:
            @pl.when(k > 0)
            def _():
                wait_prev_scatter()
        else:
            wait_prev_scatter()
        wait_idx(i + 1, t1)
        pltpu.async_copy(z_hbm.at[gbuf[t1]], rows[1 - b], gsem[1 - b])
        scale(b, t)
        pltpu.async_copy(rows[b], acc_sh.at[dbuf[t]], ssem[b], add=True)
        fire_idx(i + 2, t2)

    def block(k, _):
        for u in range(6):
            step(k * 6 + u, k, u % 2, u % 3, u)
        return 0
    lax.fori_loop(0, NCT // 6, block, 0)

    pltpu.make_async_copy(z_hbm.at[gbuf[NCT % 3]], rows[0], gsem[0]).wait()
    pltpu.make_async_copy(rows[1], acc_sh.at[dbuf[(NCT - 1) % 3]], ssem[1]).wait()
    wait_idx(NCT + 1, (NCT + 1) % 3)

    @pl.when(wid < TAIL)
    def _():
        off = (NW * NCT + wid) * CK
        pltpu.sync_copy(gidx_hbm.at[pl.ds(off, CK)], gbuf[0])
        pltpu.sync_copy(dst_hbm.at[pl.ds(off, CK)], dbuf[0])
        pltpu.sync_copy(w_hbm.at[pl.ds(off, CK)], wbuf[0])
        pltpu.async_copy(z_hbm.at[gbuf[0]], rows[0], gsem[0]).wait()
        scale(0, 0)
        pltpu.sync_copy(rows[0], acc_sh.at[dbuf[0]], add=True)

    plsc.subcore_barrier()
    ob = cid * N
    def writeout(i, _):
        pltpu.sync_copy(acc_sh.at[pl.ds(sid * ZPT + i * CK, CK)],
                        out_hbm.at[pl.ds(ob + sid * ZPT + i * CK, CK)])
        return 0
    lax.fori_loop(0, ZPT // CK, writeout, 0)
    pltpu.sync_copy(acc_sh.at[pl.ds(sid * ZPT + (ZPT // CK) * CK, ZPT % CK)],
                    out_hbm.at[pl.ds(ob + sid * ZPT + (ZPT // CK) * CK, ZPT % CK)])
    @pl.when(sid == 0)
    def _():
        pltpu.sync_copy(acc_sh.at[pl.ds(NS * ZPT, N - NS * ZPT)],
                        out_hbm.at[pl.ds(ob + NS * ZPT, N - NS * ZPT)])


def _prep_tc_body(comp0_ref, b0_ref, comp1_ref, b1_ref, g_ref, be_ref, m_ref,
                  v_ref, w0_ref, w1_ref, ab_ref):
    w0_ref[...] = jnp.dot(comp0_ref[...], b0_ref[...],
                          preferred_element_type=jnp.float32)
    w1_ref[...] = jnp.dot(comp1_ref[...], b1_ref[...],
                          preferred_element_type=jnp.float32)
    a = g_ref[...] * lax.rsqrt(v_ref[...] + EPS_BN)
    ab_ref[0:1, :] = a
    ab_ref[1:2, :] = be_ref[...] - m_ref[...] * a


_prep_tc = pl.pallas_call(
    _prep_tc_body,
    out_shape=(
        jax.ShapeDtypeStruct((R, C * C), jnp.float32),
        jax.ShapeDtypeStruct((R, C * C), jnp.float32),
        jax.ShapeDtypeStruct((2, C), jnp.float32),
    ),
)


def _proj_tc_body(h_ref, w_ref, root_ref, bias_ref, z_ref, base_ref):
    h = h_ref[...]
    for r in range(R):
        z_ref[r] = jnp.dot(h, w_ref[r], preferred_element_type=jnp.float32)
    base_ref[...] = jnp.dot(h, root_ref[...],
                            preferred_element_type=jnp.float32) + bias_ref[...]


_proj_tc = pl.pallas_call(
    _proj_tc_body,
    grid=(GRID,),
    in_specs=[
        pl.BlockSpec((NBK, C), lambda i: (i, 0)),
        pl.BlockSpec((R, C, C), lambda i: (0, 0, 0)),
        pl.BlockSpec((C, C), lambda i: (0, 0)),
        pl.BlockSpec((1, C), lambda i: (0, 0)),
    ],
    out_specs=(
        pl.BlockSpec((R, NBK, C), lambda i: (0, i, 0)),
        pl.BlockSpec((NBK, C), lambda i: (i, 0)),
    ),
    out_shape=(
        jax.ShapeDtypeStruct((R, N, C), jnp.float32),
        jax.ShapeDtypeStruct((N, C), jnp.float32),
    ),
)


def _proj2_tc_body(base0_ref, p0_ref, p1_ref, ab_ref, w_ref, root_ref,
                   bias_ref, z_ref, base_ref):
    y = base0_ref[...] + p0_ref[...] + p1_ref[...]
    h = jnp.maximum(y * ab_ref[0:1, :] + ab_ref[1:2, :], 0.0)
    for r in range(R):
        z_ref[r] = jnp.dot(h, w_ref[r], preferred_element_type=jnp.float32)
    base_ref[...] = jnp.dot(h, root_ref[...],
                            preferred_element_type=jnp.float32) + bias_ref[...]


_proj2_tc = pl.pallas_call(
    _proj2_tc_body,
    grid=(GRID,),
    in_specs=[
        pl.BlockSpec((NBK, C), lambda i: (i, 0)),
        pl.BlockSpec((NBK, C), lambda i: (i, 0)),
        pl.BlockSpec((NBK, C), lambda i: (i, 0)),
        pl.BlockSpec((2, C), lambda i: (0, 0)),
        pl.BlockSpec((R, C, C), lambda i: (0, 0, 0)),
        pl.BlockSpec((C, C), lambda i: (0, 0)),
        pl.BlockSpec((1, C), lambda i: (0, 0)),
    ],
    out_specs=(
        pl.BlockSpec((R, NBK, C), lambda i: (0, i, 0)),
        pl.BlockSpec((NBK, C), lambda i: (i, 0)),
    ),
    out_shape=(
        jax.ShapeDtypeStruct((R, N, C), jnp.float32),
        jax.ShapeDtypeStruct((N, C), jnp.float32),
    ),
)


def _final_tc_body(base_ref, p0_ref, p1_ref, out_ref):
    y = base_ref[...] + p0_ref[...] + p1_ref[...]
    nrm = jnp.sqrt(jnp.sum(y * y, axis=-1, keepdims=True))
    out_ref[...] = y / jnp.maximum(nrm, EPS_NORM)


_final_tc = pl.pallas_call(
    _final_tc_body,
    grid=(GRID,),
    in_specs=[
        pl.BlockSpec((NBK, C), lambda i: (i, 0)),
        pl.BlockSpec((NBK, C), lambda i: (i, 0)),
        pl.BlockSpec((NBK, C), lambda i: (i, 0)),
    ],
    out_specs=pl.BlockSpec((NBK, C), lambda i: (i, 0)),
    out_shape=jax.ShapeDtypeStruct((N, C), jnp.float32),
)


def kernel(x, edge_index, edge_type, comp0, bases0, root0, bias0,
           bn_gamma, bn_beta, bn_mean, bn_var, comp1, bases1, root1, bias1):
    src = edge_index[0].astype(jnp.int32)
    dst = edge_index[1].astype(jnp.int32)
    rt = edge_type.astype(jnp.int32)

    w0f, w1f, ab = _prep_tc(comp0, bases0.reshape(NB, C * C),
                            comp1, bases1.reshape(NB, C * C),
                            bn_gamma.reshape(1, C), bn_beta.reshape(1, C),
                            bn_mean.reshape(1, C), bn_var.reshape(1, C))
    w0 = w0f.reshape(R, C, C)
    w1 = w1f.reshape(R, C, C)

    cnt, gidx, cidx = _counts_sc(src, rt, dst)
    rc = _rc_tc(cnt.reshape(2 * CPAD // C, C)).reshape(CPAD)
    w = _weights_sc(rc, cidx)

    z0, base0 = _proj_tc(x, w0, root0, bias0.reshape(1, C))
    parts0 = _agg_sc(z0.reshape(R * N, C), gidx, dst, w)

    z1, base1 = _proj2_tc(base0, parts0[:N], parts0[N:], ab, w1, root1,
                          bias1.reshape(1, C))
    parts1 = _agg_sc(z1.reshape(R * N, C), gidx, dst, w)

    return _final_tc(base1, parts1[:N], parts1[N:])


# speedup vs baseline: 24.6486x; 1.0524x over previous
import functools

import jax
import jax.numpy as jnp
from jax import lax
from jax.experimental import pallas as pl
from jax.experimental.pallas import tpu as pltpu
from jax.experimental.pallas import tpu_sc as plsc

N = 10000
E = 320000
C = 128
R = 5
NB = 4
EPS_BN = 1e-5
EPS_NORM = 1e-12

NC = 2
NS = 16
L = 16
NW = NC * NS
CK = 128
NCHG = E // CK
NCT = NCHG // NW
TAIL = NCHG - NW * NCT
CPAD = 51200
CPT = CPAD // NS
ZPT = 624
NBK = 1000
GRID = N // NBK

_mesh = plsc.VectorSubcoreMesh(core_axis_name="c", subcore_axis_name="s")
_sc_params = pltpu.CompilerParams(needs_layout_passes=False)


@functools.partial(
    pl.kernel,
    out_type=(
        jax.ShapeDtypeStruct((2 * CPAD,), jnp.float32),
        jax.ShapeDtypeStruct((E,), jnp.int32),
        jax.ShapeDtypeStruct((E,), jnp.int32),
    ),
    mesh=_mesh,
    compiler_params=_sc_params,
    scratch_types=[
        pltpu.VMEM((2, CK), jnp.int32),
        pltpu.VMEM((2, CK), jnp.int32),
        pltpu.VMEM((2, CK), jnp.int32),
        pltpu.VMEM((CK,), jnp.int32),
        pltpu.VMEM((CK,), jnp.int32),
        pltpu.VMEM((CK,), jnp.int32),
        pltpu.VMEM((CK,), jnp.int32),
        pltpu.VMEM((CK,), jnp.float32),
        pltpu.VMEM((CPT,), jnp.float32),
        pltpu.SemaphoreType.DMA,
        pltpu.SemaphoreType.DMA,
        pltpu.SemaphoreType.DMA,
        pltpu.SemaphoreType.DMA,
        pltpu.VMEM_SHARED((CPAD,), jnp.float32),
    ],
)
def _counts_sc(src_hbm, rt_hbm, dst_hbm, cnt_hbm, gidx_hbm, cidx_hbm,
               src_v, rt_v, dst_v, g0, g1, c0, c1, ones_v, zer_v,
               isem0, isem1, wsem0, wsem1, acc_sh):
    cid = lax.axis_index("c")
    sid = lax.axis_index("s")
    wid = sid * NC + cid
    gbuf = (g0, g1)
    cbuf = (c0, c1)
    isem = (isem0, isem1)
    wsem = (wsem0, wsem1)

    def fill_ones(i, _):
        ones_v[pl.ds(i * L, L)] = jnp.full((L,), 1.0, jnp.float32)
        return 0
    lax.fori_loop(0, CK // L, fill_ones, 0)

    def fill_zeros(i, _):
        zer_v[pl.ds(i * L, L)] = jnp.zeros((L,), jnp.float32)
        return 0
    lax.fori_loop(0, CPT // L, fill_zeros, 0)

    pltpu.sync_copy(zer_v, acc_sh.at[pl.ds(sid * CPT, CPT)])
    plsc.subcore_barrier()

    def cg(i):
        return (wid + jnp.minimum(i, NCT - 1) * NW) * CK

    def fire_idx(i, b):
        off = cg(i)
        pltpu.async_copy(src_hbm.at[pl.ds(off, CK)], src_v.at[b], isem[b])
        pltpu.async_copy(rt_hbm.at[pl.ds(off, CK)], rt_v.at[b], isem[b])
        pltpu.async_copy(dst_hbm.at[pl.ds(off, CK)], dst_v.at[b], isem[b])

    def wait_idx(i, b):
        off = cg(i)
        pltpu.make_async_copy(src_hbm.at[pl.ds(off, CK)], src_v.at[b], isem[b]).wait()
        pltpu.make_async_copy(rt_hbm.at[pl.ds(off, CK)], rt_v.at[b], isem[b]).wait()
        pltpu.make_async_copy(dst_hbm.at[pl.ds(off, CK)], dst_v.at[b], isem[b]).wait()

    def wait_writes(i, b):
        off = cg(i)
        pltpu.make_async_copy(gbuf[b], gidx_hbm.at[pl.ds(off, CK)], wsem[b]).wait()
        pltpu.make_async_copy(cbuf[b], cidx_hbm.at[pl.ds(off, CK)], wsem[b]).wait()

    def compute(b):
        for j in range(CK // L):
            sl = pl.ds(j * L, L)
            s16 = src_v[b, sl]
            r16 = rt_v[b, sl]
            d16 = dst_v[b, sl]
            gbuf[b][sl] = r16 * N + s16
            cbuf[b][sl] = d16 * R + r16

    def step(i, b, first):
        wait_idx(i, b)
        fire_idx(i + 1, 1 - b)
        if first:
            pass
        else:
            @pl.when(i >= 2)
            def _():
                wait_writes(i - 2, b)
        compute(b)
        off = cg(i)
        pltpu.async_copy(gbuf[b], gidx_hbm.at[pl.ds(off, CK)], wsem[b])
        pltpu.async_copy(cbuf[b], cidx_hbm.at[pl.ds(off, CK)], wsem[b])
        pltpu.sync_copy(ones_v, acc_sh.at[cbuf[b]], add=True)

    fire_idx(0, 0)

    def pair(k, _):
        i = k * 2
        step(i, 0, False)
        step(i + 1, 1, False)
        return 0
    lax.fori_loop(0, NCT // 2, pair, 0)

    wait_idx(NCT, 0)
    wait_writes(NCT - 2, 0)
    wait_writes(NCT - 1, 1)

    @pl.when(wid < TAIL)
    def _():
        off = (NW * NCT + wid) * CK
        pltpu.sync_copy(src_hbm.at[pl.ds(off, CK)], src_v.at[0])
        pltpu.sync_copy(rt_hbm.at[pl.ds(off, CK)], rt_v.at[0])
        pltpu.sync_copy(dst_hbm.at[pl.ds(off, CK)], dst_v.at[0])
        compute(0)
        pltpu.sync_copy(gbuf[0], gidx_hbm.at[pl.ds(off, CK)])
        pltpu.sync_copy(cbuf[0], cidx_hbm.at[pl.ds(off, CK)])
        pltpu.sync_copy(ones_v, acc_sh.at[cbuf[0]], add=True)

    plsc.subcore_barrier()
    pltpu.sync_copy(acc_sh.at[pl.ds(sid * CPT, CPT)],
                    cnt_hbm.at[pl.ds(cid * CPAD + sid * CPT, CPT)])


def _rc_tc_body(cnt_ref, rc_ref):
    c = cnt_ref[0:CPAD // C, :] + cnt_ref[CPAD // C:, :]
    rc_ref[...] = 1.0 / jnp.maximum(c, 1.0)


_rc_tc = pl.pallas_call(
    _rc_tc_body,
    out_shape=jax.ShapeDtypeStruct((CPAD // C, C), jnp.float32),
)


CW = 512
NCWG = E // CW
NCW = NCWG // NW
WTAIL = NCWG - NW * NCW


@functools.partial(
    pl.kernel,
    out_type=jax.ShapeDtypeStruct((E,), jnp.float32),
    mesh=_mesh,
    compiler_params=_sc_params,
    scratch_types=[
        pltpu.VMEM((CPAD,), jnp.float32),
        pltpu.VMEM((2, CW), jnp.int32),
        pltpu.VMEM((CW,), jnp.float32),
        pltpu.VMEM((CW,), jnp.float32),
        pltpu.SemaphoreType.DMA,
        pltpu.SemaphoreType.DMA,
        pltpu.SemaphoreType.DMA,
        pltpu.SemaphoreType.DMA,
    ],
)
def _weights_sc(rc_hbm, cidx_hbm, w_hbm, rc_v, ci_v, w0, w1,
                isem0, isem1, wsem0, wsem1):
    wid = lax.axis_index("s") * NC + lax.axis_index("c")
    wbuf = (w0, w1)
    isem = (isem0, isem1)
    wsem = (wsem0, wsem1)

    def cgw(i):
        return (wid + min(i, NCW - 1) * NW) * CW

    pltpu.async_copy(cidx_hbm.at[pl.ds(cgw(0), CW)], ci_v.at[0], isem[0])
    pltpu.sync_copy(rc_hbm, rc_v)

    def gathers(b):
        def gbody(j, _):
            sl = pl.ds(j * L, L)
            wbuf[b][sl] = plsc.load_gather(rc_v, [ci_v[b, sl]])
            return 0
        lax.fori_loop(0, CW // L, gbody, 0)

    def step(i, b):
        off = cgw(i)
        pltpu.make_async_copy(cidx_hbm.at[pl.ds(off, CW)], ci_v.at[b], isem[b]).wait()
        pltpu.async_copy(cidx_hbm.at[pl.ds(cgw(i + 1), CW)], ci_v.at[1 - b], isem[1 - b])
        if i >= 2:
            pltpu.make_async_copy(wbuf[b], w_hbm.at[pl.ds(cgw(i - 2), CW)],
                                  wsem[b]).wait()
        gathers(b)
        pltpu.async_copy(wbuf[b], w_hbm.at[pl.ds(off, CW)], wsem[b])

    for i in range(NCW):
        step(i, i % 2)

    pltpu.make_async_copy(cidx_hbm.at[pl.ds(cgw(NCW - 1), CW)],
                          ci_v.at[1], isem[1]).wait()
    pltpu.make_async_copy(wbuf[1], w_hbm.at[pl.ds(cgw(NCW - 2), CW)],
                          wsem[1]).wait()
    pltpu.make_async_copy(wbuf[0], w_hbm.at[pl.ds(cgw(NCW - 1), CW)],
                          wsem[0]).wait()

    @pl.when(wid < WTAIL)
    def _():
        off = (NW * NCW + wid) * CW
        pltpu.sync_copy(cidx_hbm.at[pl.ds(off, CW)], ci_v.at[0])
        gathers(0)
        pltpu.sync_copy(wbuf[0], w_hbm.at[pl.ds(off, CW)])


@functools.partial(
    pl.kernel,
    out_type=jax.ShapeDtypeStruct((2 * N, C), jnp.float32),
    mesh=_mesh,
    compiler_params=_sc_params,
    scratch_types=[
        pltpu.VMEM((CK,), jnp.int32),
        pltpu.VMEM((CK,), jnp.int32),
        pltpu.VMEM((CK,), jnp.int32),
        pltpu.VMEM((CK,), jnp.int32),
        pltpu.VMEM((CK,), jnp.int32),
        pltpu.VMEM((CK,), jnp.int32),
        pltpu.VMEM((CK,), jnp.float32),
        pltpu.VMEM((CK,), jnp.float32),
        pltpu.VMEM((CK,), jnp.float32),
        pltpu.VMEM((CK, C), jnp.float32),
        pltpu.VMEM((CK, C), jnp.float32),
        pltpu.SemaphoreType.DMA,
        pltpu.SemaphoreType.DMA,
        pltpu.SemaphoreType.DMA,
        pltpu.SemaphoreType.DMA,
        pltpu.SemaphoreType.DMA,
        pltpu.SemaphoreType.DMA,
        pltpu.SemaphoreType.DMA,
        pltpu.VMEM_SHARED((N, C), jnp.float32),
    ],
)
def _agg_sc(z_hbm, gidx_hbm, dst_hbm, w_hbm, out_hbm,
            g0, g1, g2, d0, d1, d2, w0, w1, w2, rows0, rows1,
            isem0, isem1, isem2, gsem0, gsem1, ssem0, ssem1, acc_sh):
    cid = lax.axis_index("c")
    sid = lax.axis_index("s")
    wid = sid * NC + cid
    gbuf = (g0, g1, g2)
    dbuf = (d0, d1, d2)
    wbuf = (w0, w1, w2)
    rows = (rows0, rows1)
    isem = (isem0, isem1, isem2)
    gsem = (gsem0, gsem1)
    ssem = (ssem0, ssem1)

    def zero_rows(i, _):
        for j in range(C // L):
            rows0[i, pl.ds(j * L, L)] = jnp.zeros((L,), jnp.float32)
        return 0
    lax.fori_loop(0, CK, zero_rows, 0)

    zb = sid * ZPT
    def zero_acc(i, _):
        pltpu.sync_copy(rows0, acc_sh.at[pl.ds(zb + i * CK, CK)])
        return 0
    lax.fori_loop(0, ZPT // CK, zero_acc, 0)
    pltpu.sync_copy(rows0.at[pl.ds(0, ZPT % CK)],
                    acc_sh.at[pl.ds(zb + (ZPT // CK) * CK, ZPT % CK)])
    @pl.when(sid == 0)
    def _():
        pltpu.sync_copy(rows0.at[pl.ds(0, N - NS * ZPT)],
                        acc_sh.at[pl.ds(NS * ZPT, N - NS * ZPT)])
    plsc.subcore_barrier()

    def cg(i):
        return (wid + jnp.minimum(i, NCT - 1) * NW) * CK

    def fire_idx(i, t):
        off = cg(i)
        pltpu.async_copy(gidx_hbm.at[pl.ds(off, CK)], gbuf[t], isem[t])
        pltpu.async_copy(dst_hbm.at[pl.ds(off, CK)], dbuf[t], isem[t])
        pltpu.async_copy(w_hbm.at[pl.ds(off, CK)], wbuf[t], isem[t])

    def wait_idx(i, t):
        off = cg(i)
        pltpu.make_async_copy(gidx_hbm.at[pl.ds(off, CK)], gbuf[t], isem[t]).wait()
        pltpu.make_async_copy(dst_hbm.at[pl.ds(off, CK)], dbuf[t], isem[t]).wait()
        pltpu.make_async_copy(w_hbm.at[pl.ds(off, CK)], wbuf[t], isem[t]).wait()

    def scale(b, t):
        def body(h, _):
            e0 = h * 2
            e1 = h * 2 + 1
            wa = plsc.load_gather(wbuf[t], [jnp.full((L,), 0, jnp.int32) + e0])
            wb = plsc.load_gather(wbuf[t], [jnp.full((L,), 0, jnp.int32) + e1])
            for j in range(C // L):
                sl = pl.ds(j * L, L)
                rows[b][e0, sl] = rows[b][e0, sl] * wa
                rows[b][e1, sl] = rows[b][e1, sl] * wb
            return 0
        lax.fori_loop(0, CK // 2, body, 0)

    fire_idx(0, 0)
    fire_idx(1, 1)
    wait_idx(0, 0)
    pltpu.async_copy(z_hbm.at[gbuf[0]], rows[0], gsem[0])

    def step(i, k, b, t, u):
        t1 = (t + 1) % 3
        t2 = (t + 2) % 3
        pltpu.make_async_copy(z_hbm.at[gbuf[t]], rows[b], gsem[b]).wait()
        def wait_prev_scatter():
            pltpu.make_async_copy(rows[1 - b], acc_sh.at[dbuf[t2]],
                                  ssem[1 - b]).wait()
        if u == 0:
            @pl.when(k > 0)
            def _():
                wait_prev_scatter()
        else:
            wait_prev_scatter()
        wait_idx(i + 1, t1)
        pltpu.async_copy(z_hbm.at[gbuf[t1]], rows[1 - b], gsem[1 - b])
        scale(b, t)
        pltpu.async_copy(rows[b], acc_sh.at[dbuf[t]], ssem[b], add=True)
        fire_idx(i + 2, t2)

    def block(k, _):
        for u in range(6):
            step(k * 6 + u, k, u % 2, u % 3, u)
        return 0
    lax.fori_loop(0, NCT // 6, block, 0)

    pltpu.make_async_copy(z_hbm.at[gbuf[NCT % 3]], rows[0], gsem[0]).wait()
    pltpu.make_async_copy(rows[1], acc_sh.at[dbuf[(NCT - 1) % 3]], ssem[1]).wait()
    wait_idx(NCT + 1, (NCT + 1) % 3)

    @pl.when(wid < TAIL)
    def _():
        off = (NW * NCT + wid) * CK
        pltpu.sync_copy(gidx_hbm.at[pl.ds(off, CK)], gbuf[0])
        pltpu.sync_copy(dst_hbm.at[pl.ds(off, CK)], dbuf[0])
        pltpu.sync_copy(w_hbm.at[pl.ds(off, CK)], wbuf[0])
        pltpu.async_copy(z_hbm.at[gbuf[0]], rows[0], gsem[0]).wait()
        scale(0, 0)
        pltpu.sync_copy(rows[0], acc_sh.at[dbuf[0]], add=True)

    plsc.subcore_barrier()
    ob = cid * N
    def writeout(i, _):
        pltpu.sync_copy(acc_sh.at[pl.ds(sid * ZPT + i * CK, CK)],
                        out_hbm.at[pl.ds(ob + sid * ZPT + i * CK, CK)])
        return 0
    lax.fori_loop(0, ZPT // CK, writeout, 0)
    pltpu.sync_copy(acc_sh.at[pl.ds(sid * ZPT + (ZPT // CK) * CK, ZPT % CK)],
                    out_hbm.at[pl.ds(ob + sid * ZPT + (ZPT // CK) * CK, ZPT % CK)])
    @pl.when(sid == 0)
    def _():
        pltpu.sync_copy(acc_sh.at[pl.ds(NS * ZPT, N - NS * ZPT)],
                        out_hbm.at[pl.ds(ob + NS * ZPT, N - NS * ZPT)])


def _prep_tc_body(comp0_ref, b0_ref, comp1_ref, b1_ref, g_ref, be_ref, m_ref,
                  v_ref, w0_ref, w1_ref, ab_ref):
    w0_ref[...] = jnp.dot(comp0_ref[...], b0_ref[...],
                          preferred_element_type=jnp.float32)
    w1_ref[...] = jnp.dot(comp1_ref[...], b1_ref[...],
                          preferred_element_type=jnp.float32)
    a = g_ref[...] * lax.rsqrt(v_ref[...] + EPS_BN)
    ab_ref[0:1, :] = a
    ab_ref[1:2, :] = be_ref[...] - m_ref[...] * a


_prep_tc = pl.pallas_call(
    _prep_tc_body,
    out_shape=(
        jax.ShapeDtypeStruct((R, C * C), jnp.float32),
        jax.ShapeDtypeStruct((R, C * C), jnp.float32),
        jax.ShapeDtypeStruct((2, C), jnp.float32),
    ),
)


def _proj_tc_body(h_ref, w_ref, root_ref, bias_ref, z_ref, base_ref):
    h = h_ref[...]
    for r in range(R):
        z_ref[r] = jnp.dot(h, w_ref[r], preferred_element_type=jnp.float32)
    base_ref[...] = jnp.dot(h, root_ref[...],
                            preferred_element_type=jnp.float32) + bias_ref[...]


_proj_tc = pl.pallas_call(
    _proj_tc_body,
    grid=(GRID,),
    in_specs=[
        pl.BlockSpec((NBK, C), lambda i: (i, 0)),
        pl.BlockSpec((R, C, C), lambda i: (0, 0, 0)),
        pl.BlockSpec((C, C), lambda i: (0, 0)),
        pl.BlockSpec((1, C), lambda i: (0, 0)),
    ],
    out_specs=(
        pl.BlockSpec((R, NBK, C), lambda i: (0, i, 0)),
        pl.BlockSpec((NBK, C), lambda i: (i, 0)),
    ),
    out_shape=(
        jax.ShapeDtypeStruct((R, N, C), jnp.float32),
        jax.ShapeDtypeStruct((N, C), jnp.float32),
    ),
)


def _proj2_tc_body(base0_ref, p0_ref, p1_ref, ab_ref, w_ref, root_ref,
                   bias_ref, z_ref, base_ref):
    y = base0_ref[...] + p0_ref[...] + p1_ref[...]
    h = jnp.maximum(y * ab_ref[0:1, :] + ab_ref[1:2, :], 0.0)
    for r in range(R):
        z_ref[r] = jnp.dot(h, w_ref[r], preferred_element_type=jnp.float32)
    base_ref[...] = jnp.dot(h, root_ref[...],
                            preferred_element_type=jnp.float32) + bias_ref[...]


_proj2_tc = pl.pallas_call(
    _proj2_tc_body,
    grid=(GRID,),
    in_specs=[
        pl.BlockSpec((NBK, C), lambda i: (i, 0)),
        pl.BlockSpec((NBK, C), lambda i: (i, 0)),
        pl.BlockSpec((NBK, C), lambda i: (i, 0)),
        pl.BlockSpec((2, C), lambda i: (0, 0)),
        pl.BlockSpec((R, C, C), lambda i: (0, 0, 0)),
        pl.BlockSpec((C, C), lambda i: (0, 0)),
        pl.BlockSpec((1, C), lambda i: (0, 0)),
    ],
    out_specs=(
        pl.BlockSpec((R, NBK, C), lambda i: (0, i, 0)),
        pl.BlockSpec((NBK, C), lambda i: (i, 0)),
    ),
    out_shape=(
        jax.ShapeDtypeStruct((R, N, C), jnp.float32),
        jax.ShapeDtypeStruct((N, C), jnp.float32),
    ),
)


def _final_tc_body(base_ref, p0_ref, p1_ref, out_ref):
    y = base_ref[...] + p0_ref[...] + p1_ref[...]
    nrm = jnp.sqrt(jnp.sum(y * y, axis=-1, keepdims=True))
    out_ref[...] = y / jnp.maximum(nrm, EPS_NORM)


_final_tc = pl.pallas_call(
    _final_tc_body,
    grid=(GRID,),
    in_specs=[
        pl.BlockSpec((NBK, C), lambda i: (i, 0)),
        pl.BlockSpec((NBK, C), lambda i: (i, 0)),
        pl.BlockSpec((NBK, C), lambda i: (i, 0)),
    ],
    out_specs=pl.BlockSpec((NBK, C), lambda i: (i, 0)),
    out_shape=jax.ShapeDtypeStruct((N, C), jnp.float32),
)


def kernel(x, edge_index, edge_type, comp0, bases0, root0, bias0,
           bn_gamma, bn_beta, bn_mean, bn_var, comp1, bases1, root1, bias1):
    src = edge_index[0].astype(jnp.int32)
    dst = edge_index[1].astype(jnp.int32)
    rt = edge_type.astype(jnp.int32)

    w0f, w1f, ab = _prep_tc(comp0, bases0.reshape(NB, C * C),
                            comp1, bases1.reshape(NB, C * C),
                            bn_gamma.reshape(1, C), bn_beta.reshape(1, C),
                            bn_mean.reshape(1, C), bn_var.reshape(1, C))
    w0 = w0f.reshape(R, C, C)
    w1 = w1f.reshape(R, C, C)

    cnt, gidx, cidx = _counts_sc(src, rt, dst)
    rc = _rc_tc(cnt.reshape(2 * CPAD // C, C)).reshape(CPAD)
    w = _weights_sc(rc, cidx)

    z0, base0 = _proj_tc(x, w0, root0, bias0.reshape(1, C))
    parts0 = _agg_sc(z0.reshape(R * N, C), gidx, dst, w)

    z1, base1 = _proj2_tc(base0, parts0[:N], parts0[N:], ab, w1, root1,
                          bias1.reshape(1, C))
    parts1 = _agg_sc(z1.reshape(R * N, C), gidx, dst, w)

    return _final_tc(base1, parts1[:N], parts1[N:])


# speedup vs baseline: 26.1617x; 1.0614x over previous
import functools

import jax
import jax.numpy as jnp
from jax import lax
from jax.experimental import pallas as pl
from jax.experimental.pallas import tpu as pltpu
from jax.experimental.pallas import tpu_sc as plsc

N = 10000
E = 320000
C = 128
R = 5
NB = 4
EPS_BN = 1e-5
EPS_NORM = 1e-12

NC = 2
NS = 16
L = 16
NW = NC * NS
CK = 128
NCHG = E // CK
NCT = NCHG // NW
TAIL = NCHG - NW * NCT
CPAD = 51200
CPT = CPAD // NS
ZPT = 624
NBK = 1000
GRID = N // NBK

_mesh = plsc.VectorSubcoreMesh(core_axis_name="c", subcore_axis_name="s")
_sc_params = pltpu.CompilerParams(needs_layout_passes=False)


CW = 512
CWR = CW // 128
NCWG = E // CW
NCW = NCWG // NW
WTAIL = NCWG - NW * NCW


@functools.partial(
    pl.kernel,
    out_type=(
        jax.ShapeDtypeStruct((2 * CPAD,), jnp.float32),
        jax.ShapeDtypeStruct((E,), jnp.int32),
        jax.ShapeDtypeStruct((E // 128, 128), jnp.int32),
    ),
    mesh=_mesh,
    compiler_params=_sc_params,
    scratch_types=[
        pltpu.VMEM((2, CW), jnp.int32),
        pltpu.VMEM((2, CW), jnp.int32),
        pltpu.VMEM((2, CW), jnp.int32),
        pltpu.VMEM((2, CW), jnp.int32),
        pltpu.VMEM((2, CWR, 128), jnp.int32),
        pltpu.VMEM((128,), jnp.float32),
        pltpu.VMEM((CPT,), jnp.float32),
        pltpu.SemaphoreType.DMA,
        pltpu.SemaphoreType.DMA,
        pltpu.SemaphoreType.DMA,
        pltpu.SemaphoreType.DMA,
        pltpu.SemaphoreType.DMA,
        pltpu.SemaphoreType.DMA,
        pltpu.VMEM_SHARED((CPAD,), jnp.float32),
    ],
)
def _counts_sc(src_hbm, rt_hbm, dst_hbm, cnt_hbm, gidx_hbm, cidx_hbm,
               src_v, rt_v, dst_v, g_v, c_v, ones_v, zer_v,
               isem0, isem1, wsem0, wsem1, ssem0, ssem1, acc_sh):
    cid = lax.axis_index("c")
    sid = lax.axis_index("s")
    wid = sid * NC + cid
    isem = (isem0, isem1)
    wsem = (wsem0, wsem1)
    ssem = (ssem0, ssem1)

    def fill_ones(i, _):
        ones_v[pl.ds(i * L, L)] = jnp.full((L,), 1.0, jnp.float32)
        return 0
    lax.fori_loop(0, 128 // L, fill_ones, 0)

    def fill_zeros(i, _):
        zer_v[pl.ds(i * L, L)] = jnp.zeros((L,), jnp.float32)
        return 0
    lax.fori_loop(0, CPT // L, fill_zeros, 0)

    pltpu.sync_copy(zer_v, acc_sh.at[pl.ds(sid * CPT, CPT)])
    plsc.subcore_barrier()

    def cgw(i):
        return (wid + jnp.minimum(i, NCW - 1) * NW) * CW

    def cgr(i):
        return (wid + jnp.minimum(i, NCW - 1) * NW) * CWR

    def fire_idx(i, b):
        off = cgw(i)
        pltpu.async_copy(src_hbm.at[pl.ds(off, CW)], src_v.at[b], isem[b])
        pltpu.async_copy(rt_hbm.at[pl.ds(off, CW)], rt_v.at[b], isem[b])
        pltpu.async_copy(dst_hbm.at[pl.ds(off, CW)], dst_v.at[b], isem[b])

    def wait_idx(i, b):
        off = cgw(i)
        pltpu.make_async_copy(src_hbm.at[pl.ds(off, CW)], src_v.at[b], isem[b]).wait()
        pltpu.make_async_copy(rt_hbm.at[pl.ds(off, CW)], rt_v.at[b], isem[b]).wait()
        pltpu.make_async_copy(dst_hbm.at[pl.ds(off, CW)], dst_v.at[b], isem[b]).wait()

    def wait_chunk(i, b):
        pltpu.make_async_copy(g_v.at[b], gidx_hbm.at[pl.ds(cgw(i), CW)],
                              wsem[b]).wait()
        pltpu.make_async_copy(c_v.at[b], cidx_hbm.at[pl.ds(cgr(i), CWR)],
                              wsem[b]).wait()
        for r in range(CWR):
            pltpu.make_async_copy(ones_v, acc_sh.at[c_v.at[b, r]],
                                  ssem[b]).wait()

    def compute(b):
        for j in range(CW // L):
            sl = pl.ds(j * L, L)
            s16 = src_v[b, sl]
            r16 = rt_v[b, sl]
            d16 = dst_v[b, sl]
            g_v[b, sl] = r16 * N + s16
            c_v[b, j // 8, pl.ds((j % 8) * L, L)] = d16 * R + r16

    def step(i, b):
        wait_idx(i, b)
        fire_idx(i + 1, 1 - b)
        @pl.when(i >= 2)
        def _():
            wait_chunk(i - 2, b)
        compute(b)
        pltpu.async_copy(g_v.at[b], gidx_hbm.at[pl.ds(cgw(i), CW)], wsem[b])
        pltpu.async_copy(c_v.at[b], cidx_hbm.at[pl.ds(cgr(i), CWR)], wsem[b])
        for r in range(CWR):
            pltpu.async_copy(ones_v, acc_sh.at[c_v.at[b, r]], ssem[b], add=True)

    fire_idx(0, 0)

    def pair(k, _):
        i = k * 2
        step(i, 0)
        step(i + 1, 1)
        return 0
    lax.fori_loop(0, (NCW - 1) // 2, pair, 0)
    step(NCW - 1, 0)

    wait_idx(NCW, 1)
    wait_chunk(NCW - 2, 1)
    wait_chunk(NCW - 1, 0)

    @pl.when(wid < WTAIL)
    def _():
        off = (NW * NCW + wid) * CW
        pltpu.sync_copy(src_hbm.at[pl.ds(off, CW)], src_v.at[0])
        pltpu.sync_copy(rt_hbm.at[pl.ds(off, CW)], rt_v.at[0])
        pltpu.sync_copy(dst_hbm.at[pl.ds(off, CW)], dst_v.at[0])
        compute(0)
        pltpu.sync_copy(g_v.at[0], gidx_hbm.at[pl.ds(off, CW)])
        pltpu.sync_copy(c_v.at[0], cidx_hbm.at[pl.ds((NW * NCW + wid) * CWR, CWR)])
        for r in range(CWR):
            pltpu.sync_copy(ones_v, acc_sh.at[c_v.at[0, r]], add=True)

    plsc.subcore_barrier()
    pltpu.sync_copy(acc_sh.at[pl.ds(sid * CPT, CPT)],
                    cnt_hbm.at[pl.ds(cid * CPAD + sid * CPT, CPT)])


def _rc_tc_body(cnt_ref, rc_ref):
    c = cnt_ref[0:CPAD // C, :] + cnt_ref[CPAD // C:, :]
    rc_ref[...] = 1.0 / jnp.maximum(c, 1.0)


_rc_tc = pl.pallas_call(
    _rc_tc_body,
    out_shape=jax.ShapeDtypeStruct((CPAD // C, C), jnp.float32),
)


@functools.partial(
    pl.kernel,
    out_type=jax.ShapeDtypeStruct((E,), jnp.float32),
    mesh=_mesh,
    compiler_params=_sc_params,
    scratch_types=[
        pltpu.VMEM((CPAD,), jnp.float32),
        pltpu.VMEM((2, CW), jnp.int32),
        pltpu.VMEM((CW,), jnp.float32),
        pltpu.VMEM((CW,), jnp.float32),
        pltpu.SemaphoreType.DMA,
        pltpu.SemaphoreType.DMA,
        pltpu.SemaphoreType.DMA,
        pltpu.SemaphoreType.DMA,
    ],
)
def _weights_sc(rc_hbm, cidx_hbm, w_hbm, rc_v, ci_v, w0, w1,
                isem0, isem1, wsem0, wsem1):
    wid = lax.axis_index("s") * NC + lax.axis_index("c")
    wbuf = (w0, w1)
    isem = (isem0, isem1)
    wsem = (wsem0, wsem1)

    def cgw(i):
        return (wid + min(i, NCW - 1) * NW) * CW

    pltpu.async_copy(cidx_hbm.at[pl.ds(cgw(0), CW)], ci_v.at[0], isem[0])
    pltpu.sync_copy(rc_hbm, rc_v)

    def gathers(b):
        def gbody(j, _):
            sl = pl.ds(j * L, L)
            wbuf[b][sl] = plsc.load_gather(rc_v, [ci_v[b, sl]])
            return 0
        lax.fori_loop(0, CW // L, gbody, 0)

    def step(i, b):
        off = cgw(i)
        pltpu.make_async_copy(cidx_hbm.at[pl.ds(off, CW)], ci_v.at[b], isem[b]).wait()
        pltpu.async_copy(cidx_hbm.at[pl.ds(cgw(i + 1), CW)], ci_v.at[1 - b], isem[1 - b])
        if i >= 2:
            pltpu.make_async_copy(wbuf[b], w_hbm.at[pl.ds(cgw(i - 2), CW)],
                                  wsem[b]).wait()
        gathers(b)
        pltpu.async_copy(wbuf[b], w_hbm.at[pl.ds(off, CW)], wsem[b])

    for i in range(NCW):
        step(i, i % 2)

    pltpu.make_async_copy(cidx_hbm.at[pl.ds(cgw(NCW - 1), CW)],
                          ci_v.at[1], isem[1]).wait()
    pltpu.make_async_copy(wbuf[1], w_hbm.at[pl.ds(cgw(NCW - 2), CW)],
                          wsem[1]).wait()
    pltpu.make_async_copy(wbuf[0], w_hbm.at[pl.ds(cgw(NCW - 1), CW)],
                          wsem[0]).wait()

    @pl.when(wid < WTAIL)
    def _():
        off = (NW * NCW + wid) * CW
        pltpu.sync_copy(cidx_hbm.at[pl.ds(off, CW)], ci_v.at[0])
        gathers(0)
        pltpu.sync_copy(wbuf[0], w_hbm.at[pl.ds(off, CW)])


@functools.partial(
    pl.kernel,
    out_type=jax.ShapeDtypeStruct((2 * N, C), jnp.float32),
    mesh=_mesh,
    compiler_params=_sc_params,
    scratch_types=[
        pltpu.VMEM((CK,), jnp.int32),
        pltpu.VMEM((CK,), jnp.int32),
        pltpu.VMEM((CK,), jnp.int32),
        pltpu.VMEM((CK,), jnp.int32),
        pltpu.VMEM((CK,), jnp.int32),
        pltpu.VMEM((CK,), jnp.int32),
        pltpu.VMEM((CK,), jnp.float32),
        pltpu.VMEM((CK,), jnp.float32),
        pltpu.VMEM((CK,), jnp.float32),
        pltpu.VMEM((CK, C), jnp.float32),
        pltpu.VMEM((CK, C), jnp.float32),
        pltpu.SemaphoreType.DMA,
        pltpu.SemaphoreType.DMA,
        pltpu.SemaphoreType.DMA,
        pltpu.SemaphoreType.DMA,
        pltpu.SemaphoreType.DMA,
        pltpu.SemaphoreType.DMA,
        pltpu.SemaphoreType.DMA,
        pltpu.VMEM_SHARED((N, C), jnp.float32),
    ],
)
def _agg_sc(z_hbm, gidx_hbm, dst_hbm, w_hbm, out_hbm,
            g0, g1, g2, d0, d1, d2, w0, w1, w2, rows0, rows1,
            isem0, isem1, isem2, gsem0, gsem1, ssem0, ssem1, acc_sh):
    cid = lax.axis_index("c")
    sid = lax.axis_index("s")
    wid = sid * NC + cid
    gbuf = (g0, g1, g2)
    dbuf = (d0, d1, d2)
    wbuf = (w0, w1, w2)
    rows = (rows0, rows1)
    isem = (isem0, isem1, isem2)
    gsem = (gsem0, gsem1)
    ssem = (ssem0, ssem1)

    def zero_rows(i, _):
        for j in range(C // L):
            rows0[i, pl.ds(j * L, L)] = jnp.zeros((L,), jnp.float32)
        return 0
    lax.fori_loop(0, CK, zero_rows, 0)

    zb = sid * ZPT
    def zero_acc(i, _):
        pltpu.sync_copy(rows0, acc_sh.at[pl.ds(zb + i * CK, CK)])
        return 0
    lax.fori_loop(0, ZPT // CK, zero_acc, 0)
    pltpu.sync_copy(rows0.at[pl.ds(0, ZPT % CK)],
                    acc_sh.at[pl.ds(zb + (ZPT // CK) * CK, ZPT % CK)])
    @pl.when(sid == 0)
    def _():
        pltpu.sync_copy(rows0.at[pl.ds(0, N - NS * ZPT)],
                        acc_sh.at[pl.ds(NS * ZPT, N - NS * ZPT)])
    plsc.subcore_barrier()

    def cg(i):
        return (wid + jnp.minimum(i, NCT - 1) * NW) * CK

    def fire_idx(i, t):
        off = cg(i)
        pltpu.async_copy(gidx_hbm.at[pl.ds(off, CK)], gbuf[t], isem[t])
        pltpu.async_copy(dst_hbm.at[pl.ds(off, CK)], dbuf[t], isem[t])
        pltpu.async_copy(w_hbm.at[pl.ds(off, CK)], wbuf[t], isem[t])

    def wait_idx(i, t):
        off = cg(i)
        pltpu.make_async_copy(gidx_hbm.at[pl.ds(off, CK)], gbuf[t], isem[t]).wait()
        pltpu.make_async_copy(dst_hbm.at[pl.ds(off, CK)], dbuf[t], isem[t]).wait()
        pltpu.make_async_copy(w_hbm.at[pl.ds(off, CK)], wbuf[t], isem[t]).wait()

    def scale(b, t):
        def body(h, _):
            e0 = h * 2
            e1 = h * 2 + 1
            wa = plsc.load_gather(wbuf[t], [jnp.full((L,), 0, jnp.int32) + e0])
            wb = plsc.load_gather(wbuf[t], [jnp.full((L,), 0, jnp.int32) + e1])
            for j in range(C // L):
                sl = pl.ds(j * L, L)
                rows[b][e0, sl] = rows[b][e0, sl] * wa
                rows[b][e1, sl] = rows[b][e1, sl] * wb
            return 0
        lax.fori_loop(0, CK // 2, body, 0)

    fire_idx(0, 0)
    fire_idx(1, 1)
    wait_idx(0, 0)
    pltpu.async_copy(z_hbm.at[gbuf[0]], rows[0], gsem[0])

    def step(i, k, b, t, u):
        t1 = (t + 1) % 3
        t2 = (t + 2) % 3
        pltpu.make_async_copy(z_hbm.at[gbuf[t]], rows[b], gsem[b]).wait()
        def wait_prev_scatter():
            pltpu.make_async_copy(rows[1 - b], acc_sh.at[dbuf[t2]],
                                  ssem[1 - b]).wait()
        if u == 0:
            @pl.when(k > 0)
            def _():
                wait_prev_scatter()
        else:
            wait_prev_scatter()
        wait_idx(i + 1, t1)
        pltpu.async_copy(z_hbm.at[gbuf[t1]], rows[1 - b], gsem[1 - b])
        scale(b, t)
        pltpu.async_copy(rows[b], acc_sh.at[dbuf[t]], ssem[b], add=True)
        fire_idx(i + 2, t2)

    def block(k, _):
        for u in range(6):
            step(k * 6 + u, k, u % 2, u % 3, u)
        return 0
    lax.fori_loop(0, NCT // 6, block, 0)

    pltpu.make_async_copy(z_hbm.at[gbuf[NCT % 3]], rows[0], gsem[0]).wait()
    pltpu.make_async_copy(rows[1], acc_sh.at[dbuf[(NCT - 1) % 3]], ssem[1]).wait()
    wait_idx(NCT + 1, (NCT + 1) % 3)

    @pl.when(wid < TAIL)
    def _():
        off = (NW * NCT + wid) * CK
        pltpu.sync_copy(gidx_hbm.at[pl.ds(off, CK)], gbuf[0])
        pltpu.sync_copy(dst_hbm.at[pl.ds(off, CK)], dbuf[0])
        pltpu.sync_copy(w_hbm.at[pl.ds(off, CK)], wbuf[0])
        pltpu.async_copy(z_hbm.at[gbuf[0]], rows[0], gsem[0]).wait()
        scale(0, 0)
        pltpu.sync_copy(rows[0], acc_sh.at[dbuf[0]], add=True)

    plsc.subcore_barrier()
    ob = cid * N
    def writeout(i, _):
        pltpu.sync_copy(acc_sh.at[pl.ds(sid * ZPT + i * CK, CK)],
                        out_hbm.at[pl.ds(ob + sid * ZPT + i * CK, CK)])
        return 0
    lax.fori_loop(0, ZPT // CK, writeout, 0)
    pltpu.sync_copy(acc_sh.at[pl.ds(sid * ZPT + (ZPT // CK) * CK, ZPT % CK)],
                    out_hbm.at[pl.ds(ob + sid * ZPT + (ZPT // CK) * CK, ZPT % CK)])
    @pl.when(sid == 0)
    def _():
        pltpu.sync_copy(acc_sh.at[pl.ds(NS * ZPT, N - NS * ZPT)],
                        out_hbm.at[pl.ds(ob + NS * ZPT, N - NS * ZPT)])


def _prep_tc_body(comp0_ref, b0_ref, comp1_ref, b1_ref, g_ref, be_ref, m_ref,
                  v_ref, w0_ref, w1_ref, ab_ref):
    w0_ref[...] = jnp.dot(comp0_ref[...], b0_ref[...],
                          preferred_element_type=jnp.float32)
    w1_ref[...] = jnp.dot(comp1_ref[...], b1_ref[...],
                          preferred_element_type=jnp.float32)
    a = g_ref[...] * lax.rsqrt(v_ref[...] + EPS_BN)
    ab_ref[0:1, :] = a
    ab_ref[1:2, :] = be_ref[...] - m_ref[...] * a


_prep_tc = pl.pallas_call(
    _prep_tc_body,
    out_shape=(
        jax.ShapeDtypeStruct((R, C * C), jnp.float32),
        jax.ShapeDtypeStruct((R, C * C), jnp.float32),
        jax.ShapeDtypeStruct((2, C), jnp.float32),
    ),
)


def _proj_tc_body(h_ref, w_ref, root_ref, bias_ref, z_ref, base_ref):
    h = h_ref[...]
    for r in range(R):
        z_ref[r] = jnp.dot(h, w_ref[r], preferred_element_type=jnp.float32)
    base_ref[...] = jnp.dot(h, root_ref[...],
                            preferred_element_type=jnp.float32) + bias_ref[...]


_proj_tc = pl.pallas_call(
    _proj_tc_body,
    grid=(GRID,),
    in_specs=[
        pl.BlockSpec((NBK, C), lambda i: (i, 0)),
        pl.BlockSpec((R, C, C), lambda i: (0, 0, 0)),
        pl.BlockSpec((C, C), lambda i: (0, 0)),
        pl.BlockSpec((1, C), lambda i: (0, 0)),
    ],
    out_specs=(
        pl.BlockSpec((R, NBK, C), lambda i: (0, i, 0)),
        pl.BlockSpec((NBK, C), lambda i: (i, 0)),
    ),
    out_shape=(
        jax.ShapeDtypeStruct((R, N, C), jnp.float32),
        jax.ShapeDtypeStruct((N, C), jnp.float32),
    ),
)


def _proj2_tc_body(base0_ref, p0_ref, p1_ref, ab_ref, w_ref, root_ref,
                   bias_ref, z_ref, base_ref):
    y = base0_ref[...] + p0_ref[...] + p1_ref[...]
    h = jnp.maximum(y * ab_ref[0:1, :] + ab_ref[1:2, :], 0.0)
    for r in range(R):
        z_ref[r] = jnp.dot(h, w_ref[r], preferred_element_type=jnp.float32)
    base_ref[...] = jnp.dot(h, root_ref[...],
                            preferred_element_type=jnp.float32) + bias_ref[...]


_proj2_tc = pl.pallas_call(
    _proj2_tc_body,
    grid=(GRID,),
    in_specs=[
        pl.BlockSpec((NBK, C), lambda i: (i, 0)),
        pl.BlockSpec((NBK, C), lambda i: (i, 0)),
        pl.BlockSpec((NBK, C), lambda i: (i, 0)),
        pl.BlockSpec((2, C), lambda i: (0, 0)),
        pl.BlockSpec((R, C, C), lambda i: (0, 0, 0)),
        pl.BlockSpec((C, C), lambda i: (0, 0)),
        pl.BlockSpec((1, C), lambda i: (0, 0)),
    ],
    out_specs=(
        pl.BlockSpec((R, NBK, C), lambda i: (0, i, 0)),
        pl.BlockSpec((NBK, C), lambda i: (i, 0)),
    ),
    out_shape=(
        jax.ShapeDtypeStruct((R, N, C), jnp.float32),
        jax.ShapeDtypeStruct((N, C), jnp.float32),
    ),
)


def _final_tc_body(base_ref, p0_ref, p1_ref, out_ref):
    y = base_ref[...] + p0_ref[...] + p1_ref[...]
    nrm = jnp.sqrt(jnp.sum(y * y, axis=-1, keepdims=True))
    out_ref[...] = y / jnp.maximum(nrm, EPS_NORM)


_final_tc = pl.pallas_call(
    _final_tc_body,
    grid=(GRID,),
    in_specs=[
        pl.BlockSpec((NBK, C), lambda i: (i, 0)),
        pl.BlockSpec((NBK, C), lambda i: (i, 0)),
        pl.BlockSpec((NBK, C), lambda i: (i, 0)),
    ],
    out_specs=pl.BlockSpec((NBK, C), lambda i: (i, 0)),
    out_shape=jax.ShapeDtypeStruct((N, C), jnp.float32),
)


def kernel(x, edge_index, edge_type, comp0, bases0, root0, bias0,
           bn_gamma, bn_beta, bn_mean, bn_var, comp1, bases1, root1, bias1):
    src = edge_index[0].astype(jnp.int32)
    dst = edge_index[1].astype(jnp.int32)
    rt = edge_type.astype(jnp.int32)

    w0f, w1f, ab = _prep_tc(comp0, bases0.reshape(NB, C * C),
                            comp1, bases1.reshape(NB, C * C),
                            bn_gamma.reshape(1, C), bn_beta.reshape(1, C),
                            bn_mean.reshape(1, C), bn_var.reshape(1, C))
    w0 = w0f.reshape(R, C, C)
    w1 = w1f.reshape(R, C, C)

    cnt, gidx, cidx2 = _counts_sc(src, rt, dst)
    rc = _rc_tc(cnt.reshape(2 * CPAD // C, C)).reshape(CPAD)
    w = _weights_sc(rc, cidx2.reshape(E))

    z0, base0 = _proj_tc(x, w0, root0, bias0.reshape(1, C))
    parts0 = _agg_sc(z0.reshape(R * N, C), gidx, dst, w)

    z1, base1 = _proj2_tc(base0, parts0[:N], parts0[N:], ab, w1, root1,
                          bias1.reshape(1, C))
    parts1 = _agg_sc(z1.reshape(R * N, C), gidx, dst, w)

    return _final_tc(base1, parts1[:N], parts1[N:])
